# Initial kernel scaffold; baseline (speedup 1.0000x reference)
import jax
import jax.numpy as jnp
from jax.experimental import pallas as pl


def kernel(x, edge_index, W1, b1, W2, b2, Wd, bd, perm_ids):
    raise NotImplementedError("write your pallas kernel here")


# speedup vs baseline: 17.1615x; 17.1615x over previous
import functools

import jax
import jax.numpy as jnp
from jax import lax
from jax.experimental import pallas as pl
from jax.experimental.pallas import tpu as pltpu
from jax.experimental.pallas import tpu_sc as plsc

N = 10000
E = 320000
D = 128
NC = 2
NS = 16
CH = 125
NCH = E // NS // CH
RPT = 640
HROW = 640
HN = NS * HROW
PC = 5


def _tile_row_start(s):
  return pl.multiple_of(jnp.where(s == NS - 1, N - RPT, s * RPT), 8)

_mesh = lambda: plsc.VectorSubcoreMesh(
    core_axis_name="c", subcore_axis_name="s", num_cores=NC, num_subcores=NS)


def _sc_hist_perm(didx, sidx, perm, xW1, ones_h, zeros_h):
  @functools.partial(
      pl.kernel,
      out_type=(
          jax.ShapeDtypeStruct((NC, HN), jnp.float32),
          jax.ShapeDtypeStruct((N, D), jnp.float32),
      ),
      mesh=_mesh(),
      scratch_types=[
          pltpu.VMEM((NCH, CH), jnp.int32),
          pltpu.VMEM((CH,), jnp.float32),
          pltpu.VMEM((RPT,), jnp.int32),
          pltpu.VMEM((128, D), jnp.float32),
          pltpu.VMEM_SHARED((HN,), jnp.float32),
          pltpu.SemaphoreType.DMA,
      ],
  )
  def k(didx_h, sidx_h, perm_h, xw_h, ones_hb, zeros_hb, hist_o, p0_o,
        iv, onesv, pv, rbuf, acc1, sem):
    c = lax.axis_index("c")
    s = lax.axis_index("s")
    pltpu.sync_copy(zeros_hb, acc1.at[pl.ds(s * HROW, HROW)])
    pltpu.sync_copy(ones_hb, onesv)

    @pl.when(c == 0)
    def _():
      pltpu.sync_copy(didx_h.at[s], iv)

    @pl.when(c == 1)
    def _():
      pltpu.sync_copy(sidx_h.at[s], iv)

    plsc.subcore_barrier()

    def body(j, carry):
      pltpu.sync_copy(onesv, acc1.at[iv.at[j]], add=True)
      return carry

    lax.fori_loop(0, NCH, body, 0)
    plsc.subcore_barrier()
    pltpu.sync_copy(acc1.at[pl.ds(s * HROW, HROW)],
                    hist_o.at[c, pl.ds(s * HROW, HROW)])

    @pl.when(c == 1)
    def _():
      start = _tile_row_start(s)
      pltpu.sync_copy(perm_h.at[pl.ds(start, RPT)], pv)
      def pbody(kk, carry):
        pltpu.async_copy(xw_h.at[pv.at[pl.ds(kk * 128, 128)]], rbuf,
                         sem).wait()
        pltpu.sync_copy(rbuf,
                        p0_o.at[pl.ds(pl.multiple_of(start + kk * 128, 8),
                                      128)])
        return carry
      lax.fori_loop(0, PC, pbody, 0)

  return k(didx, sidx, perm, xW1, ones_h, zeros_h)


def _sc_dual_pass(A0, A1, init0, init1, ecat, ranges):
  (st0, cnt0), (st1, cnt1) = ranges

  @functools.partial(
      pl.kernel,
      out_type=jax.ShapeDtypeStruct((NC, N, D), jnp.float32),
      mesh=_mesh(),
      scratch_types=[
          pltpu.VMEM((2, CH), jnp.int32),
          pltpu.VMEM((2, CH), jnp.int32),
          pltpu.VMEM((CH, D), jnp.float32),
          pltpu.VMEM((CH, D), jnp.float32),
          pltpu.VMEM_SHARED((N, D), jnp.float32),
          pltpu.SemaphoreType.DMA,
          pltpu.SemaphoreType.DMA,
          pltpu.SemaphoreType.DMA,
          pltpu.SemaphoreType.DMA,
      ],
  )
  def k(a0_h, a1_h, i0_h, i1_h, ecat_h, out_o,
        ib0, ib1, b0, b1, acc, sg0, sg1, si0, si1):
    c = lax.axis_index("c")
    s = lax.axis_index("s")
    start = _tile_row_start(s)

    @pl.when(c == 0)
    def _():
      pltpu.sync_copy(i0_h.at[pl.ds(start, RPT)], acc.at[pl.ds(start, RPT)])

    @pl.when(c == 1)
    def _():
      pltpu.sync_copy(i1_h.at[pl.ds(start, RPT)], acc.at[pl.ds(start, RPT)])

    plsc.subcore_barrier()

    def run(a_h, st, cnt):
      end = st + cnt
      pltpu.sync_copy(ecat_h.at[s, st], ib0)
      pltpu.async_copy(a_h.at[ib0.at[0]], b0, sg0)
      pltpu.async_copy(ecat_h.at[s, st + 1], ib1, si1)

      def body(kk, carry):
        j0 = st + 2 * kk
        pltpu.make_async_copy(ecat_h.at[s, j0 + 1], ib1, si1).wait()
        pltpu.async_copy(a_h.at[ib1.at[0]], b1, sg1)
        pltpu.make_async_copy(a_h.at[ib0.at[0]], b0, sg0).wait()
        pltpu.sync_copy(b0, acc.at[ib0.at[1]], add=True)

        @pl.when(j0 + 2 < end)
        def _():
          pltpu.async_copy(ecat_h.at[s, j0 + 2], ib0, si0)
          pltpu.make_async_copy(ecat_h.at[s, j0 + 2], ib0, si0).wait()
          pltpu.async_copy(a_h.at[ib0.at[0]], b0, sg0)

        pltpu.make_async_copy(a_h.at[ib1.at[0]], b1, sg1).wait()
        pltpu.sync_copy(b1, acc.at[ib1.at[1]], add=True)

        @pl.when(j0 + 3 < end)
        def _():
          pltpu.async_copy(ecat_h.at[s, j0 + 3], ib1, si1)

        return carry

      lax.fori_loop(0, cnt // 2, body, 0)

    @pl.when(c == 0)
    def _():
      run(a0_h, st0, cnt0)

    @pl.when(c == 1)
    def _():
      run(a1_h, st1, cnt1)

    plsc.subcore_barrier()
    pltpu.sync_copy(acc.at[pl.ds(start, RPT)],
                    out_o.at[c, pl.ds(start, RPT)])

  return k(A0, A1, init0, init1, ecat)


_BLK = 2000


def _row_specs(*widths):
  return [pl.BlockSpec((_BLK, w), lambda i, _w=None: (i, 0)) for w in widths]


def _tc_matmul(x, W):
  def f(x_ref, w_ref, o_ref):
    o_ref[...] = jnp.dot(x_ref[...], w_ref[...],
                         preferred_element_type=jnp.float32)

  return pl.pallas_call(
      f,
      grid=(N // _BLK,),
      in_specs=[
          pl.BlockSpec((_BLK, D), lambda i: (i, 0)),
          pl.BlockSpec((D, D), lambda i: (0, 0)),
      ],
      out_specs=pl.BlockSpec((_BLK, D), lambda i: (i, 0)),
      out_shape=jax.ShapeDtypeStruct((N, D), jnp.float32),
  )(x, W)


def _tc_matmul_scale(x, W, scale):
  def f(x_ref, w_ref, s_ref, o_ref):
    o_ref[...] = jnp.dot(x_ref[...], w_ref[...],
                         preferred_element_type=jnp.float32) * s_ref[...]

  return pl.pallas_call(
      f,
      grid=(N // _BLK,),
      in_specs=[
          pl.BlockSpec((_BLK, D), lambda i: (i, 0)),
          pl.BlockSpec((D, D), lambda i: (0, 0)),
          pl.BlockSpec((_BLK, 1), lambda i: (i, 0)),
      ],
      out_specs=pl.BlockSpec((_BLK, D), lambda i: (i, 0)),
      out_shape=jax.ShapeDtypeStruct((N, D), jnp.float32),
  )(x, W, scale)


def _tc_prescale(hist_d, hist_s, xW1, P0):
  def f(hd_ref, hsr_ref, xw_ref, p0_ref, dis_ref, ci_ref, hs1_ref, hsa_ref):
    deg = hd_ref[...] + 1.0
    dis = lax.rsqrt(deg)
    cnt = hsr_ref[...]
    ci_ref[...] = 1.0 / jnp.where(cnt == 0.0, 1.0, cnt)
    dis_ref[...] = dis
    hs1_ref[...] = xw_ref[...] * dis
    hsa_ref[...] = p0_ref[...] * dis

  return pl.pallas_call(
      f,
      grid=(N // _BLK,),
      in_specs=[
          pl.BlockSpec((_BLK, 1), lambda i: (i, 0)),
          pl.BlockSpec((_BLK, 1), lambda i: (i, 0)),
          pl.BlockSpec((_BLK, D), lambda i: (i, 0)),
          pl.BlockSpec((_BLK, D), lambda i: (i, 0)),
      ],
      out_specs=[
          pl.BlockSpec((_BLK, 1), lambda i: (i, 0)),
          pl.BlockSpec((_BLK, 1), lambda i: (i, 0)),
          pl.BlockSpec((_BLK, D), lambda i: (i, 0)),
          pl.BlockSpec((_BLK, D), lambda i: (i, 0)),
      ],
      out_shape=[
          jax.ShapeDtypeStruct((N, 1), jnp.float32),
          jax.ShapeDtypeStruct((N, 1), jnp.float32),
          jax.ShapeDtypeStruct((N, D), jnp.float32),
          jax.ShapeDtypeStruct((N, D), jnp.float32),
      ],
  )(hist_d, hist_s, xW1, P0)


def _tc_conv_epilogue(acc0, acc1, dis, b):
  def f(a0_ref, a1_ref, dis_ref, b_ref, z_ref, za_ref):
    d = dis_ref[...]
    bb = b_ref[...]
    z_ref[...] = jnp.maximum(a0_ref[...] * d + bb, 0.0)
    za_ref[...] = jnp.maximum(a1_ref[...] * d + bb, 0.0)

  return pl.pallas_call(
      f,
      grid=(N // _BLK,),
      in_specs=[
          pl.BlockSpec((_BLK, D), lambda i: (i, 0)),
          pl.BlockSpec((_BLK, D), lambda i: (i, 0)),
          pl.BlockSpec((_BLK, 1), lambda i: (i, 0)),
          pl.BlockSpec((1, D), lambda i: (0, 0)),
      ],
      out_specs=[
          pl.BlockSpec((_BLK, D), lambda i: (i, 0)),
          pl.BlockSpec((_BLK, D), lambda i: (i, 0)),
      ],
      out_shape=[
          jax.ShapeDtypeStruct((N, D), jnp.float32),
          jax.ShapeDtypeStruct((N, D), jnp.float32),
      ],
  )(acc0, acc1, dis, b)


def _tc_final_epilogue(c20, c21, r0, r1, cntinv, dis, b2):
  def f(c20_ref, c21_ref, r0_ref, r1_ref, ci_ref, dis_ref, b_ref,
        h_ref, g_ref, ga_ref):
    h_ref[...] = jnp.maximum(
        (c20_ref[...] + c21_ref[...]) * dis_ref[...] + b_ref[...], 0.0)

    def readout(r):
      gr = r * ci_ref[...]
      nrm = jnp.sqrt(jnp.sum(gr * gr, axis=1, keepdims=True))
      gr = gr / jnp.maximum(nrm, 1e-12)
      return 1.0 / (1.0 + jnp.exp(-gr))

    g_ref[...] = readout(r0_ref[...])
    ga_ref[...] = readout(r1_ref[...])

  return pl.pallas_call(
      f,
      grid=(N // _BLK,),
      in_specs=[
          pl.BlockSpec((_BLK, D), lambda i: (i, 0)),
          pl.BlockSpec((_BLK, D), lambda i: (i, 0)),
          pl.BlockSpec((_BLK, D), lambda i: (i, 0)),
          pl.BlockSpec((_BLK, D), lambda i: (i, 0)),
          pl.BlockSpec((_BLK, 1), lambda i: (i, 0)),
          pl.BlockSpec((_BLK, 1), lambda i: (i, 0)),
          pl.BlockSpec((1, D), lambda i: (0, 0)),
      ],
      out_specs=[
          pl.BlockSpec((_BLK, D), lambda i: (i, 0)),
          pl.BlockSpec((_BLK, D), lambda i: (i, 0)),
          pl.BlockSpec((_BLK, D), lambda i: (i, 0)),
      ],
      out_shape=[
          jax.ShapeDtypeStruct((N, D), jnp.float32),
          jax.ShapeDtypeStruct((N, D), jnp.float32),
          jax.ShapeDtypeStruct((N, D), jnp.float32),
      ],
  )(c20, c21, r0, r1, cntinv, dis, b2)


def _tc_discriminator(g, g_a, z, z_a, Wd0, bd):
  def f(g_ref, ga_ref, z_ref, za_ref, w_ref, bd_ref, ret_ref, reta_ref):
    wg = lax.dot_general(g_ref[...], w_ref[...],
                         (((1,), (1,)), ((), ())),
                         preferred_element_type=jnp.float32)
    wga = lax.dot_general(ga_ref[...], w_ref[...],
                          (((1,), (1,)), ((), ())),
                          preferred_element_type=jnp.float32)
    b = bd_ref[0, 0]
    s1 = jnp.sum(z_ref[...] * wg, axis=1, keepdims=True)
    s2 = jnp.sum(za_ref[...] * wg, axis=1, keepdims=True)
    ret_ref[...] = jnp.concatenate([s1, s2], axis=1) + b
    s3 = jnp.sum(za_ref[...] * wga, axis=1, keepdims=True)
    s4 = jnp.sum(z_ref[...] * wga, axis=1, keepdims=True)
    reta_ref[...] = jnp.concatenate([s3, s4], axis=1) + b

  return pl.pallas_call(
      f,
      grid=(N // _BLK,),
      in_specs=[
          pl.BlockSpec((_BLK, D), lambda i: (i, 0)),
          pl.BlockSpec((_BLK, D), lambda i: (i, 0)),
          pl.BlockSpec((_BLK, D), lambda i: (i, 0)),
          pl.BlockSpec((_BLK, D), lambda i: (i, 0)),
          pl.BlockSpec((D, D), lambda i: (0, 0)),
          pl.BlockSpec((1, 1), lambda i: (0, 0)),
      ],
      out_specs=[
          pl.BlockSpec((_BLK, 2), lambda i: (i, 0)),
          pl.BlockSpec((_BLK, 2), lambda i: (i, 0)),
      ],
      out_shape=[
          jax.ShapeDtypeStruct((N, 2), jnp.float32),
          jax.ShapeDtypeStruct((N, 2), jnp.float32),
      ],
  )(g, g_a, z, z_a, Wd0, bd)


def kernel(x, edge_index, W1, b1, W2, b2, Wd, bd, perm_ids):
  src = edge_index[0].reshape(NS, NCH, CH)
  dst = edge_index[1].reshape(NS, NCH, CH)
  e_conv = jnp.stack([src, dst], axis=2)
  e_read = jnp.stack([dst, src], axis=2)
  ones_h = jnp.ones((CH,), jnp.float32)
  zeros_h = jnp.zeros((HROW,), jnp.float32)
  zeros_nd = jnp.zeros((N, D), jnp.float32)
  b1r = b1.reshape(1, D)
  b2r = b2.reshape(1, D)
  bdr = bd.reshape(1, 1)

  xW1 = _tc_matmul(x, W1)
  hist, P0 = _sc_hist_perm(dst, src, perm_ids, xW1, ones_h, zeros_h)
  hist_d = hist[0, :N].reshape(N, 1)
  hist_s = hist[1, :N].reshape(N, 1)
  dis, cntinv, hs1, hs1a = _tc_prescale(hist_d, hist_s, xW1, P0)

  conv1 = _sc_dual_pass(hs1, hs1a, hs1, hs1a, e_conv,
                        ((0, NCH), (0, NCH)))
  z, z_a = _tc_conv_epilogue(conv1[0], conv1[1], dis, b1r)

  hs2 = _tc_matmul_scale(z, W2, dis)
  rout = _sc_dual_pass(z, z_a, zeros_nd, zeros_nd, e_read,
                       ((0, NCH), (0, NCH)))
  conv2 = _sc_dual_pass(hs2, hs2, hs2, zeros_nd, e_conv,
                        ((0, NCH // 2), (NCH // 2, NCH // 2)))

  h, g, g_a = _tc_final_epilogue(conv2[0], conv2[1], rout[0], rout[1],
                                 cntinv, dis, b2r)
  ret, ret_a = _tc_discriminator(g, g_a, z, z_a, Wd[0], bdr)
  return (z, h, ret, ret_a)


# speedup vs baseline: 19.1876x; 1.1181x over previous
import functools

import jax
import jax.numpy as jnp
from jax import lax
from jax.experimental import pallas as pl
from jax.experimental.pallas import tpu as pltpu
from jax.experimental.pallas import tpu_sc as plsc

N = 10000
E = 320000
D = 128
NC = 2
NS = 16
CH = 125
NCH = E // NS // CH
RPT = 640
HROW = 640
HN = NS * HROW
PC = 5


def _tile_row_start(s):
  return pl.multiple_of(jnp.where(s == NS - 1, N - RPT, s * RPT), 8)

_mesh = lambda: plsc.VectorSubcoreMesh(
    core_axis_name="c", subcore_axis_name="s", num_cores=NC, num_subcores=NS)


def _sc_hist_perm(didx, sidx, perm, xW1, ones_h, zeros_h):
  @functools.partial(
      pl.kernel,
      out_type=(
          jax.ShapeDtypeStruct((NC, HN), jnp.float32),
          jax.ShapeDtypeStruct((N, D), jnp.float32),
      ),
      mesh=_mesh(),
      scratch_types=[
          pltpu.VMEM((NCH, CH), jnp.int32),
          pltpu.VMEM((CH,), jnp.float32),
          pltpu.VMEM((RPT,), jnp.int32),
          pltpu.VMEM((128, D), jnp.float32),
          pltpu.VMEM_SHARED((HN,), jnp.float32),
          pltpu.SemaphoreType.DMA,
      ],
  )
  def k(didx_h, sidx_h, perm_h, xw_h, ones_hb, zeros_hb, hist_o, p0_o,
        iv, onesv, pv, rbuf, acc1, sem):
    c = lax.axis_index("c")
    s = lax.axis_index("s")
    pltpu.sync_copy(zeros_hb, acc1.at[pl.ds(s * HROW, HROW)])
    pltpu.sync_copy(ones_hb, onesv)

    @pl.when(c == 0)
    def _():
      pltpu.sync_copy(didx_h.at[s], iv)

    @pl.when(c == 1)
    def _():
      pltpu.sync_copy(sidx_h.at[s], iv)

    plsc.subcore_barrier()

    def body(j, carry):
      pltpu.sync_copy(onesv, acc1.at[iv.at[j]], add=True)
      return carry

    lax.fori_loop(0, NCH, body, 0)
    plsc.subcore_barrier()
    pltpu.sync_copy(acc1.at[pl.ds(s * HROW, HROW)],
                    hist_o.at[c, pl.ds(s * HROW, HROW)])

    @pl.when(c == 1)
    def _():
      start = _tile_row_start(s)
      pltpu.sync_copy(perm_h.at[pl.ds(start, RPT)], pv)
      def pbody(kk, carry):
        pltpu.async_copy(xw_h.at[pv.at[pl.ds(kk * 128, 128)]], rbuf,
                         sem).wait()
        pltpu.sync_copy(rbuf,
                        p0_o.at[pl.ds(pl.multiple_of(start + kk * 128, 8),
                                      128)])
        return carry
      lax.fori_loop(0, PC, pbody, 0)

  return k(didx, sidx, perm, xW1, ones_h, zeros_h)


GB = 8


def _sc_dual_pass(A0, A1, init0, init1, ecat, ranges):
  (st0, cnt0), (st1, cnt1) = ranges

  @functools.partial(
      pl.kernel,
      out_type=jax.ShapeDtypeStruct((NC, N, D), jnp.float32),
      mesh=_mesh(),
      scratch_types=[
          pltpu.VMEM((2, GB, 2, CH), jnp.int32),
          pltpu.VMEM((2, CH, D), jnp.float32),
          pltpu.VMEM_SHARED((N, D), jnp.float32),
          pltpu.SemaphoreType.DMA,
          pltpu.SemaphoreType.DMA,
          pltpu.SemaphoreType.DMA,
          pltpu.SemaphoreType.DMA,
          pltpu.SemaphoreType.DMA,
      ],
  )
  def k(a0_h, a1_h, i0_h, i1_h, ecat_h, out_o,
        ibg, bufs, acc, sg0, sg1, ss0, ss1, si):
    c = lax.axis_index("c")
    s = lax.axis_index("s")
    start = _tile_row_start(s)
    sg = (sg0, sg1)
    ss = (ss0, ss1)

    @pl.when(c == 0)
    def _():
      pltpu.sync_copy(i0_h.at[pl.ds(start, RPT)], acc.at[pl.ds(start, RPT)])

    @pl.when(c == 1)
    def _():
      pltpu.sync_copy(i1_h.at[pl.ds(start, RPT)], acc.at[pl.ds(start, RPT)])

    plsc.subcore_barrier()

    def run(a_h, st, cnt):
      ngr = cnt // GB

      def wait_scatter(slot, gs_i, u_i):
        pltpu.make_async_copy(bufs.at[slot], acc.at[ibg.at[gs_i, u_i, 1]],
                              ss[slot]).wait()

      pltpu.sync_copy(ecat_h.at[s, pl.ds(st, GB)], ibg.at[0])
      pltpu.async_copy(a_h.at[ibg.at[0, 0, 0]], bufs.at[0], sg[0])

      def group(g, carry):
        gs = g % 2
        base = st + g * GB
        for u in range(GB):
          slot = u % 2
          nslot = (u + 1) % 2
          if u < GB - 1:
            if u >= 1:
              wait_scatter(nslot, gs, u + 1)
            else:
              @pl.when(g > 0)
              def _():
                wait_scatter(nslot, gs, u + 1)
            pltpu.async_copy(a_h.at[ibg.at[gs, u + 1, 0]], bufs.at[nslot],
                             sg[nslot])
          else:
            @pl.when(g + 1 < ngr)
            def _():
              wait_scatter(nslot, 1 - gs, 0)
              pltpu.make_async_copy(ecat_h.at[s, pl.ds(base + GB, GB)],
                                    ibg.at[1 - gs], si).wait()
              pltpu.async_copy(a_h.at[ibg.at[1 - gs, 0, 0]], bufs.at[nslot],
                               sg[nslot])
          pltpu.make_async_copy(a_h.at[ibg.at[gs, u, 0]], bufs.at[slot],
                                sg[slot]).wait()
          pltpu.async_copy(bufs.at[slot], acc.at[ibg.at[gs, u, 1]], ss[slot],
                           add=True)
          if u == 3:
            @pl.when(g + 1 < ngr)
            def _():
              pltpu.async_copy(ecat_h.at[s, pl.ds(base + GB, GB)],
                               ibg.at[1 - gs], si)
        return carry

      lax.fori_loop(0, ngr, group, 0)
      for slot in range(2):
        wait_scatter(slot, 0, slot)

    @pl.when(c == 0)
    def _():
      run(a0_h, st0, cnt0)

    @pl.when(c == 1)
    def _():
      run(a1_h, st1, cnt1)

    plsc.subcore_barrier()
    pltpu.sync_copy(acc.at[pl.ds(start, RPT)],
                    out_o.at[c, pl.ds(start, RPT)])

  return k(A0, A1, init0, init1, ecat)


_BLK = 2000


def _row_specs(*widths):
  return [pl.BlockSpec((_BLK, w), lambda i, _w=None: (i, 0)) for w in widths]


def _tc_matmul(x, W):
  def f(x_ref, w_ref, o_ref):
    o_ref[...] = jnp.dot(x_ref[...], w_ref[...],
                         preferred_element_type=jnp.float32)

  return pl.pallas_call(
      f,
      grid=(N // _BLK,),
      in_specs=[
          pl.BlockSpec((_BLK, D), lambda i: (i, 0)),
          pl.BlockSpec((D, D), lambda i: (0, 0)),
      ],
      out_specs=pl.BlockSpec((_BLK, D), lambda i: (i, 0)),
      out_shape=jax.ShapeDtypeStruct((N, D), jnp.float32),
  )(x, W)


def _tc_matmul_scale(x, W, scale):
  def f(x_ref, w_ref, s_ref, o_ref):
    o_ref[...] = jnp.dot(x_ref[...], w_ref[...],
                         preferred_element_type=jnp.float32) * s_ref[...]

  return pl.pallas_call(
      f,
      grid=(N // _BLK,),
      in_specs=[
          pl.BlockSpec((_BLK, D), lambda i: (i, 0)),
          pl.BlockSpec((D, D), lambda i: (0, 0)),
          pl.BlockSpec((_BLK, 1), lambda i: (i, 0)),
      ],
      out_specs=pl.BlockSpec((_BLK, D), lambda i: (i, 0)),
      out_shape=jax.ShapeDtypeStruct((N, D), jnp.float32),
  )(x, W, scale)


def _tc_prescale(hist_d, hist_s, xW1, P0):
  def f(hd_ref, hsr_ref, xw_ref, p0_ref, dis_ref, ci_ref, hs1_ref, hsa_ref):
    deg = hd_ref[...] + 1.0
    dis = lax.rsqrt(deg)
    cnt = hsr_ref[...]
    ci_ref[...] = 1.0 / jnp.where(cnt == 0.0, 1.0, cnt)
    dis_ref[...] = dis
    hs1_ref[...] = xw_ref[...] * dis
    hsa_ref[...] = p0_ref[...] * dis

  return pl.pallas_call(
      f,
      grid=(N // _BLK,),
      in_specs=[
          pl.BlockSpec((_BLK, 1), lambda i: (i, 0)),
          pl.BlockSpec((_BLK, 1), lambda i: (i, 0)),
          pl.BlockSpec((_BLK, D), lambda i: (i, 0)),
          pl.BlockSpec((_BLK, D), lambda i: (i, 0)),
      ],
      out_specs=[
          pl.BlockSpec((_BLK, 1), lambda i: (i, 0)),
          pl.BlockSpec((_BLK, 1), lambda i: (i, 0)),
          pl.BlockSpec((_BLK, D), lambda i: (i, 0)),
          pl.BlockSpec((_BLK, D), lambda i: (i, 0)),
      ],
      out_shape=[
          jax.ShapeDtypeStruct((N, 1), jnp.float32),
          jax.ShapeDtypeStruct((N, 1), jnp.float32),
          jax.ShapeDtypeStruct((N, D), jnp.float32),
          jax.ShapeDtypeStruct((N, D), jnp.float32),
      ],
  )(hist_d, hist_s, xW1, P0)


def _tc_conv_epilogue(acc0, acc1, dis, b):
  def f(a0_ref, a1_ref, dis_ref, b_ref, z_ref, za_ref):
    d = dis_ref[...]
    bb = b_ref[...]
    z_ref[...] = jnp.maximum(a0_ref[...] * d + bb, 0.0)
    za_ref[...] = jnp.maximum(a1_ref[...] * d + bb, 0.0)

  return pl.pallas_call(
      f,
      grid=(N // _BLK,),
      in_specs=[
          pl.BlockSpec((_BLK, D), lambda i: (i, 0)),
          pl.BlockSpec((_BLK, D), lambda i: (i, 0)),
          pl.BlockSpec((_BLK, 1), lambda i: (i, 0)),
          pl.BlockSpec((1, D), lambda i: (0, 0)),
      ],
      out_specs=[
          pl.BlockSpec((_BLK, D), lambda i: (i, 0)),
          pl.BlockSpec((_BLK, D), lambda i: (i, 0)),
      ],
      out_shape=[
          jax.ShapeDtypeStruct((N, D), jnp.float32),
          jax.ShapeDtypeStruct((N, D), jnp.float32),
      ],
  )(acc0, acc1, dis, b)


def _tc_final_epilogue(c20, c21, r0, r1, cntinv, dis, b2):
  def f(c20_ref, c21_ref, r0_ref, r1_ref, ci_ref, dis_ref, b_ref,
        h_ref, g_ref, ga_ref):
    h_ref[...] = jnp.maximum(
        (c20_ref[...] + c21_ref[...]) * dis_ref[...] + b_ref[...], 0.0)

    def readout(r):
      gr = r * ci_ref[...]
      nrm = jnp.sqrt(jnp.sum(gr * gr, axis=1, keepdims=True))
      gr = gr / jnp.maximum(nrm, 1e-12)
      return 1.0 / (1.0 + jnp.exp(-gr))

    g_ref[...] = readout(r0_ref[...])
    ga_ref[...] = readout(r1_ref[...])

  return pl.pallas_call(
      f,
      grid=(N // _BLK,),
      in_specs=[
          pl.BlockSpec((_BLK, D), lambda i: (i, 0)),
          pl.BlockSpec((_BLK, D), lambda i: (i, 0)),
          pl.BlockSpec((_BLK, D), lambda i: (i, 0)),
          pl.BlockSpec((_BLK, D), lambda i: (i, 0)),
          pl.BlockSpec((_BLK, 1), lambda i: (i, 0)),
          pl.BlockSpec((_BLK, 1), lambda i: (i, 0)),
          pl.BlockSpec((1, D), lambda i: (0, 0)),
      ],
      out_specs=[
          pl.BlockSpec((_BLK, D), lambda i: (i, 0)),
          pl.BlockSpec((_BLK, D), lambda i: (i, 0)),
          pl.BlockSpec((_BLK, D), lambda i: (i, 0)),
      ],
      out_shape=[
          jax.ShapeDtypeStruct((N, D), jnp.float32),
          jax.ShapeDtypeStruct((N, D), jnp.float32),
          jax.ShapeDtypeStruct((N, D), jnp.float32),
      ],
  )(c20, c21, r0, r1, cntinv, dis, b2)


def _tc_discriminator(g, g_a, z, z_a, Wd0, bd):
  def f(g_ref, ga_ref, z_ref, za_ref, w_ref, bd_ref, ret_ref, reta_ref):
    wg = lax.dot_general(g_ref[...], w_ref[...],
                         (((1,), (1,)), ((), ())),
                         preferred_element_type=jnp.float32)
    wga = lax.dot_general(ga_ref[...], w_ref[...],
                          (((1,), (1,)), ((), ())),
                          preferred_element_type=jnp.float32)
    b = bd_ref[0, 0]
    s1 = jnp.sum(z_ref[...] * wg, axis=1, keepdims=True)
    s2 = jnp.sum(za_ref[...] * wg, axis=1, keepdims=True)
    ret_ref[...] = jnp.concatenate([s1, s2], axis=1) + b
    s3 = jnp.sum(za_ref[...] * wga, axis=1, keepdims=True)
    s4 = jnp.sum(z_ref[...] * wga, axis=1, keepdims=True)
    reta_ref[...] = jnp.concatenate([s3, s4], axis=1) + b

  return pl.pallas_call(
      f,
      grid=(N // _BLK,),
      in_specs=[
          pl.BlockSpec((_BLK, D), lambda i: (i, 0)),
          pl.BlockSpec((_BLK, D), lambda i: (i, 0)),
          pl.BlockSpec((_BLK, D), lambda i: (i, 0)),
          pl.BlockSpec((_BLK, D), lambda i: (i, 0)),
          pl.BlockSpec((D, D), lambda i: (0, 0)),
          pl.BlockSpec((1, 1), lambda i: (0, 0)),
      ],
      out_specs=[
          pl.BlockSpec((_BLK, 2), lambda i: (i, 0)),
          pl.BlockSpec((_BLK, 2), lambda i: (i, 0)),
      ],
      out_shape=[
          jax.ShapeDtypeStruct((N, 2), jnp.float32),
          jax.ShapeDtypeStruct((N, 2), jnp.float32),
      ],
  )(g, g_a, z, z_a, Wd0, bd)


def kernel(x, edge_index, W1, b1, W2, b2, Wd, bd, perm_ids):
  src = edge_index[0].reshape(NS, NCH, CH)
  dst = edge_index[1].reshape(NS, NCH, CH)
  e_conv = jnp.stack([src, dst], axis=2)
  e_read = jnp.stack([dst, src], axis=2)
  ones_h = jnp.ones((CH,), jnp.float32)
  zeros_h = jnp.zeros((HROW,), jnp.float32)
  zeros_nd = jnp.zeros((N, D), jnp.float32)
  b1r = b1.reshape(1, D)
  b2r = b2.reshape(1, D)
  bdr = bd.reshape(1, 1)

  xW1 = _tc_matmul(x, W1)
  hist, P0 = _sc_hist_perm(dst, src, perm_ids, xW1, ones_h, zeros_h)
  hist_d = hist[0, :N].reshape(N, 1)
  hist_s = hist[1, :N].reshape(N, 1)
  dis, cntinv, hs1, hs1a = _tc_prescale(hist_d, hist_s, xW1, P0)

  conv1 = _sc_dual_pass(hs1, hs1a, hs1, hs1a, e_conv,
                        ((0, NCH), (0, NCH)))
  z, z_a = _tc_conv_epilogue(conv1[0], conv1[1], dis, b1r)

  hs2 = _tc_matmul_scale(z, W2, dis)
  rout = _sc_dual_pass(z, z_a, zeros_nd, zeros_nd, e_read,
                       ((0, NCH), (0, NCH)))
  conv2 = _sc_dual_pass(hs2, hs2, hs2, zeros_nd, e_conv,
                        ((0, NCH // 2), (NCH // 2, NCH // 2)))

  h, g, g_a = _tc_final_epilogue(conv2[0], conv2[1], rout[0], rout[1],
                                 cntinv, dis, b2r)
  ret, ret_a = _tc_discriminator(g, g_a, z, z_a, Wd[0], bdr)
  return (z, h, ret, ret_a)


# speedup vs baseline: 19.8353x; 1.0338x over previous
import functools

import jax
import jax.numpy as jnp
from jax import lax
from jax.experimental import pallas as pl
from jax.experimental.pallas import tpu as pltpu
from jax.experimental.pallas import tpu_sc as plsc

N = 10000
E = 320000
D = 128
NC = 2
NS = 16
CH = 125
NCH = E // NS // CH
RPT = 640
HROW = 640
HN = NS * HROW
PC = 5


def _tile_row_start(s):
  return pl.multiple_of(jnp.where(s == NS - 1, N - RPT, s * RPT), 8)

_mesh = lambda: plsc.VectorSubcoreMesh(
    core_axis_name="c", subcore_axis_name="s", num_cores=NC, num_subcores=NS)


def _sc_hist_perm(didx, sidx, perm, xW1, ones_h, zeros_h):
  @functools.partial(
      pl.kernel,
      out_type=(
          jax.ShapeDtypeStruct((NC, HN), jnp.float32),
          jax.ShapeDtypeStruct((N, D), jnp.float32),
      ),
      mesh=_mesh(),
      scratch_types=[
          pltpu.VMEM((NCH, CH), jnp.int32),
          pltpu.VMEM((CH,), jnp.float32),
          pltpu.VMEM((RPT,), jnp.int32),
          pltpu.VMEM((128, D), jnp.float32),
          pltpu.VMEM_SHARED((HN,), jnp.float32),
          pltpu.SemaphoreType.DMA,
      ],
  )
  def k(didx_h, sidx_h, perm_h, xw_h, ones_hb, zeros_hb, hist_o, p0_o,
        iv, onesv, pv, rbuf, acc1, sem):
    c = lax.axis_index("c")
    s = lax.axis_index("s")
    pltpu.sync_copy(zeros_hb, acc1.at[pl.ds(s * HROW, HROW)])
    pltpu.sync_copy(ones_hb, onesv)

    @pl.when(c == 0)
    def _():
      pltpu.sync_copy(didx_h.at[s], iv)

    @pl.when(c == 1)
    def _():
      pltpu.sync_copy(sidx_h.at[s], iv)

    plsc.subcore_barrier()

    def body(j, carry):
      pltpu.sync_copy(onesv, acc1.at[iv.at[j]], add=True)
      return carry

    lax.fori_loop(0, NCH, body, 0)
    plsc.subcore_barrier()
    pltpu.sync_copy(acc1.at[pl.ds(s * HROW, HROW)],
                    hist_o.at[c, pl.ds(s * HROW, HROW)])

    @pl.when(c == 1)
    def _():
      start = _tile_row_start(s)
      pltpu.sync_copy(perm_h.at[pl.ds(start, RPT)], pv)
      def pbody(kk, carry):
        pltpu.async_copy(xw_h.at[pv.at[pl.ds(kk * 128, 128)]], rbuf,
                         sem).wait()
        pltpu.sync_copy(rbuf,
                        p0_o.at[pl.ds(pl.multiple_of(start + kk * 128, 8),
                                      128)])
        return carry
      lax.fori_loop(0, PC, pbody, 0)

  return k(didx, sidx, perm, xW1, ones_h, zeros_h)


GB = 8


def _sc_dual_pass(A0, A1, init0, init1, ecat, ranges):
  (st0, cnt0), (st1, cnt1) = ranges

  @functools.partial(
      pl.kernel,
      out_type=(
          jax.ShapeDtypeStruct((N, D), jnp.float32),
          jax.ShapeDtypeStruct((N, D), jnp.float32),
      ),
      mesh=_mesh(),
      scratch_types=[
          pltpu.VMEM((2, GB, 2, CH), jnp.int32),
          pltpu.VMEM((2, CH, D), jnp.float32),
          pltpu.VMEM_SHARED((N, D), jnp.float32),
          pltpu.SemaphoreType.DMA,
          pltpu.SemaphoreType.DMA,
          pltpu.SemaphoreType.DMA,
          pltpu.SemaphoreType.DMA,
          pltpu.SemaphoreType.DMA,
      ],
  )
  def k(a0_h, a1_h, i0_h, i1_h, ecat_h, out0_o, out1_o,
        ibg, bufs, acc, sg0, sg1, ss0, ss1, si):
    c = lax.axis_index("c")
    s = lax.axis_index("s")
    start = _tile_row_start(s)
    sg = (sg0, sg1)
    ss = (ss0, ss1)

    @pl.when(c == 0)
    def _():
      pltpu.sync_copy(i0_h.at[pl.ds(start, RPT)], acc.at[pl.ds(start, RPT)])

    @pl.when(c == 1)
    def _():
      pltpu.sync_copy(i1_h.at[pl.ds(start, RPT)], acc.at[pl.ds(start, RPT)])

    plsc.subcore_barrier()

    def run(a_h, st, cnt):
      ngr = cnt // GB

      def wait_scatter(slot, gs_i, u_i):
        pltpu.make_async_copy(bufs.at[slot], acc.at[ibg.at[gs_i, u_i, 1]],
                              ss[slot]).wait()

      pltpu.sync_copy(ecat_h.at[s, pl.ds(st, GB)], ibg.at[0])
      pltpu.async_copy(a_h.at[ibg.at[0, 0, 0]], bufs.at[0], sg[0])

      def group(g, carry):
        gs = g % 2
        base = st + g * GB
        for u in range(GB):
          slot = u % 2
          nslot = (u + 1) % 2
          if u < GB - 1:
            if u >= 1:
              wait_scatter(nslot, gs, u + 1)
            else:
              @pl.when(g > 0)
              def _():
                wait_scatter(nslot, gs, u + 1)
            pltpu.async_copy(a_h.at[ibg.at[gs, u + 1, 0]], bufs.at[nslot],
                             sg[nslot])
          else:
            @pl.when(g + 1 < ngr)
            def _():
              wait_scatter(nslot, 1 - gs, 0)
              pltpu.make_async_copy(ecat_h.at[s, pl.ds(base + GB, GB)],
                                    ibg.at[1 - gs], si).wait()
              pltpu.async_copy(a_h.at[ibg.at[1 - gs, 0, 0]], bufs.at[nslot],
                               sg[nslot])
          pltpu.make_async_copy(a_h.at[ibg.at[gs, u, 0]], bufs.at[slot],
                                sg[slot]).wait()
          pltpu.async_copy(bufs.at[slot], acc.at[ibg.at[gs, u, 1]], ss[slot],
                           add=True)
          if u == 3:
            @pl.when(g + 1 < ngr)
            def _():
              pltpu.async_copy(ecat_h.at[s, pl.ds(base + GB, GB)],
                               ibg.at[1 - gs], si)
        return carry

      lax.fori_loop(0, ngr, group, 0)
      for slot in range(2):
        wait_scatter(slot, 0, slot)

    @pl.when(c == 0)
    def _():
      run(a0_h, st0, cnt0)

    @pl.when(c == 1)
    def _():
      run(a1_h, st1, cnt1)

    plsc.subcore_barrier()

    @pl.when(c == 0)
    def _():
      pltpu.sync_copy(acc.at[pl.ds(start, RPT)],
                      out0_o.at[pl.ds(start, RPT)])

    @pl.when(c == 1)
    def _():
      pltpu.sync_copy(acc.at[pl.ds(start, RPT)],
                      out1_o.at[pl.ds(start, RPT)])

  return k(A0, A1, init0, init1, ecat)


_BLK = 2000


def _row_specs(*widths):
  return [pl.BlockSpec((_BLK, w), lambda i, _w=None: (i, 0)) for w in widths]


def _tc_matmul(x, W):
  def f(x_ref, w_ref, o_ref):
    o_ref[...] = jnp.dot(x_ref[...], w_ref[...],
                         preferred_element_type=jnp.float32)

  return pl.pallas_call(
      f,
      grid=(N // _BLK,),
      in_specs=[
          pl.BlockSpec((_BLK, D), lambda i: (i, 0)),
          pl.BlockSpec((D, D), lambda i: (0, 0)),
      ],
      out_specs=pl.BlockSpec((_BLK, D), lambda i: (i, 0)),
      out_shape=jax.ShapeDtypeStruct((N, D), jnp.float32),
  )(x, W)


def _tc_matmul_scale(x, W, scale):
  def f(x_ref, w_ref, s_ref, o_ref):
    o_ref[...] = jnp.dot(x_ref[...], w_ref[...],
                         preferred_element_type=jnp.float32) * s_ref[...]

  return pl.pallas_call(
      f,
      grid=(N // _BLK,),
      in_specs=[
          pl.BlockSpec((_BLK, D), lambda i: (i, 0)),
          pl.BlockSpec((D, D), lambda i: (0, 0)),
          pl.BlockSpec((_BLK, 1), lambda i: (i, 0)),
      ],
      out_specs=pl.BlockSpec((_BLK, D), lambda i: (i, 0)),
      out_shape=jax.ShapeDtypeStruct((N, D), jnp.float32),
  )(x, W, scale)


def _tc_prescale(hist_d, hist_s, xW1, P0):
  def f(hd_ref, hsr_ref, xw_ref, p0_ref, dis_ref, ci_ref, hs1_ref, hsa_ref):
    deg = hd_ref[...] + 1.0
    dis = lax.rsqrt(deg)
    cnt = hsr_ref[...]
    ci_ref[...] = 1.0 / jnp.where(cnt == 0.0, 1.0, cnt)
    dis_ref[...] = dis
    hs1_ref[...] = xw_ref[...] * dis
    hsa_ref[...] = p0_ref[...] * dis

  return pl.pallas_call(
      f,
      grid=(N // _BLK,),
      in_specs=[
          pl.BlockSpec((_BLK, 1), lambda i: (i, 0)),
          pl.BlockSpec((_BLK, 1), lambda i: (i, 0)),
          pl.BlockSpec((_BLK, D), lambda i: (i, 0)),
          pl.BlockSpec((_BLK, D), lambda i: (i, 0)),
      ],
      out_specs=[
          pl.BlockSpec((_BLK, 1), lambda i: (i, 0)),
          pl.BlockSpec((_BLK, 1), lambda i: (i, 0)),
          pl.BlockSpec((_BLK, D), lambda i: (i, 0)),
          pl.BlockSpec((_BLK, D), lambda i: (i, 0)),
      ],
      out_shape=[
          jax.ShapeDtypeStruct((N, 1), jnp.float32),
          jax.ShapeDtypeStruct((N, 1), jnp.float32),
          jax.ShapeDtypeStruct((N, D), jnp.float32),
          jax.ShapeDtypeStruct((N, D), jnp.float32),
      ],
  )(hist_d, hist_s, xW1, P0)


def _tc_conv_epilogue(acc0, acc1, dis, b):
  def f(a0_ref, a1_ref, dis_ref, b_ref, z_ref, za_ref):
    d = dis_ref[...]
    bb = b_ref[...]
    z_ref[...] = jnp.maximum(a0_ref[...] * d + bb, 0.0)
    za_ref[...] = jnp.maximum(a1_ref[...] * d + bb, 0.0)

  return pl.pallas_call(
      f,
      grid=(N // _BLK,),
      in_specs=[
          pl.BlockSpec((_BLK, D), lambda i: (i, 0)),
          pl.BlockSpec((_BLK, D), lambda i: (i, 0)),
          pl.BlockSpec((_BLK, 1), lambda i: (i, 0)),
          pl.BlockSpec((1, D), lambda i: (0, 0)),
      ],
      out_specs=[
          pl.BlockSpec((_BLK, D), lambda i: (i, 0)),
          pl.BlockSpec((_BLK, D), lambda i: (i, 0)),
      ],
      out_shape=[
          jax.ShapeDtypeStruct((N, D), jnp.float32),
          jax.ShapeDtypeStruct((N, D), jnp.float32),
      ],
  )(acc0, acc1, dis, b)


def _tc_final(c20, c21, r0, r1, z, z_a, cntinv, dis, b2, Wd0, bd):
  def f(c20_ref, c21_ref, r0_ref, r1_ref, z_ref, za_ref, ci_ref, dis_ref,
        b_ref, w_ref, bd_ref, h_ref, ret_ref, reta_ref):
    h_ref[...] = jnp.maximum(
        (c20_ref[...] + c21_ref[...]) * dis_ref[...] + b_ref[...], 0.0)

    def readout(r):
      gr = r * ci_ref[...]
      nrm = jnp.sqrt(jnp.sum(gr * gr, axis=1, keepdims=True))
      gr = gr / jnp.maximum(nrm, 1e-12)
      return 1.0 / (1.0 + jnp.exp(-gr))

    g = readout(r0_ref[...])
    g_a = readout(r1_ref[...])
    wg = lax.dot_general(g, w_ref[...], (((1,), (1,)), ((), ())),
                         preferred_element_type=jnp.float32)
    wga = lax.dot_general(g_a, w_ref[...], (((1,), (1,)), ((), ())),
                          preferred_element_type=jnp.float32)
    b = bd_ref[0, 0]
    s1 = jnp.sum(z_ref[...] * wg, axis=1, keepdims=True)
    s2 = jnp.sum(za_ref[...] * wg, axis=1, keepdims=True)
    ret_ref[...] = jnp.concatenate([s1, s2], axis=1) + b
    s3 = jnp.sum(za_ref[...] * wga, axis=1, keepdims=True)
    s4 = jnp.sum(z_ref[...] * wga, axis=1, keepdims=True)
    reta_ref[...] = jnp.concatenate([s3, s4], axis=1) + b

  return pl.pallas_call(
      f,
      grid=(N // _BLK,),
      in_specs=[
          pl.BlockSpec((_BLK, D), lambda i: (i, 0)),
          pl.BlockSpec((_BLK, D), lambda i: (i, 0)),
          pl.BlockSpec((_BLK, D), lambda i: (i, 0)),
          pl.BlockSpec((_BLK, D), lambda i: (i, 0)),
          pl.BlockSpec((_BLK, D), lambda i: (i, 0)),
          pl.BlockSpec((_BLK, D), lambda i: (i, 0)),
          pl.BlockSpec((_BLK, 1), lambda i: (i, 0)),
          pl.BlockSpec((_BLK, 1), lambda i: (i, 0)),
          pl.BlockSpec((1, D), lambda i: (0, 0)),
          pl.BlockSpec((D, D), lambda i: (0, 0)),
          pl.BlockSpec((1, 1), lambda i: (0, 0)),
      ],
      out_specs=[
          pl.BlockSpec((_BLK, D), lambda i: (i, 0)),
          pl.BlockSpec((_BLK, 2), lambda i: (i, 0)),
          pl.BlockSpec((_BLK, 2), lambda i: (i, 0)),
      ],
      out_shape=[
          jax.ShapeDtypeStruct((N, D), jnp.float32),
          jax.ShapeDtypeStruct((N, 2), jnp.float32),
          jax.ShapeDtypeStruct((N, 2), jnp.float32),
      ],
  )(c20, c21, r0, r1, z, z_a, cntinv, dis, b2, Wd0, bd)


def kernel(x, edge_index, W1, b1, W2, b2, Wd, bd, perm_ids):
  src = edge_index[0].reshape(NS, NCH, CH)
  dst = edge_index[1].reshape(NS, NCH, CH)
  e_conv = jnp.stack([src, dst], axis=2)
  e_read = jnp.stack([dst, src], axis=2)
  ones_h = jnp.ones((CH,), jnp.float32)
  zeros_h = jnp.zeros((HROW,), jnp.float32)
  zeros_nd = jnp.zeros((N, D), jnp.float32)
  b1r = b1.reshape(1, D)
  b2r = b2.reshape(1, D)
  bdr = bd.reshape(1, 1)

  xW1 = _tc_matmul(x, W1)
  hist, P0 = _sc_hist_perm(dst, src, perm_ids, xW1, ones_h, zeros_h)
  hist_d = hist[0, :N].reshape(N, 1)
  hist_s = hist[1, :N].reshape(N, 1)
  dis, cntinv, hs1, hs1a = _tc_prescale(hist_d, hist_s, xW1, P0)

  c10, c11 = _sc_dual_pass(hs1, hs1a, hs1, hs1a, e_conv,
                           ((0, NCH), (0, NCH)))
  z, z_a = _tc_conv_epilogue(c10, c11, dis, b1r)

  hs2 = _tc_matmul_scale(z, W2, dis)
  r0, r1 = _sc_dual_pass(z, z_a, zeros_nd, zeros_nd, e_read,
                         ((0, NCH), (0, NCH)))
  c20, c21 = _sc_dual_pass(hs2, hs2, hs2, zeros_nd, e_conv,
                           ((0, NCH // 2), (NCH // 2, NCH // 2)))

  h, ret, ret_a = _tc_final(c20, c21, r0, r1, z, z_a, cntinv, dis, b2r,
                            Wd[0], bdr)
  return (z, h, ret, ret_a)


# speedup vs baseline: 19.9523x; 1.0059x over previous
import functools

import jax
import jax.numpy as jnp
from jax import lax
from jax.experimental import pallas as pl
from jax.experimental.pallas import tpu as pltpu
from jax.experimental.pallas import tpu_sc as plsc

N = 10000
E = 320000
D = 128
NC = 2
NS = 16
CH = 125
NCH = E // NS // CH
RPT = 640
HROW = 640
HN = NS * HROW
PC = 5


def _tile_row_start(s):
  return pl.multiple_of(jnp.where(s == NS - 1, N - RPT, s * RPT), 8)

_mesh = lambda: plsc.VectorSubcoreMesh(
    core_axis_name="c", subcore_axis_name="s", num_cores=NC, num_subcores=NS)


def _sc_hist_perm(didx, sidx, perm, xW1, ones_h, zeros_h):
  @functools.partial(
      pl.kernel,
      out_type=(
          jax.ShapeDtypeStruct((NC, HN), jnp.float32),
          jax.ShapeDtypeStruct((N, D), jnp.float32),
      ),
      mesh=_mesh(),
      scratch_types=[
          pltpu.VMEM((NCH, CH), jnp.int32),
          pltpu.VMEM((CH,), jnp.float32),
          pltpu.VMEM((RPT,), jnp.int32),
          pltpu.VMEM((128, D), jnp.float32),
          pltpu.VMEM_SHARED((HN,), jnp.float32),
          pltpu.SemaphoreType.DMA,
      ],
  )
  def k(didx_h, sidx_h, perm_h, xw_h, ones_hb, zeros_hb, hist_o, p0_o,
        iv, onesv, pv, rbuf, acc1, sem):
    c = lax.axis_index("c")
    s = lax.axis_index("s")
    pltpu.sync_copy(zeros_hb, acc1.at[pl.ds(s * HROW, HROW)])
    pltpu.sync_copy(ones_hb, onesv)

    @pl.when(c == 0)
    def _():
      pltpu.sync_copy(didx_h.at[s], iv)

    @pl.when(c == 1)
    def _():
      pltpu.sync_copy(sidx_h.at[s], iv)

    plsc.subcore_barrier()

    def body(j, carry):
      pltpu.sync_copy(onesv, acc1.at[iv.at[j]], add=True)
      return carry

    lax.fori_loop(0, NCH, body, 0)
    plsc.subcore_barrier()
    pltpu.sync_copy(acc1.at[pl.ds(s * HROW, HROW)],
                    hist_o.at[c, pl.ds(s * HROW, HROW)])

    @pl.when(c == 1)
    def _():
      start = _tile_row_start(s)
      pltpu.sync_copy(perm_h.at[pl.ds(start, RPT)], pv)
      def pbody(kk, carry):
        pltpu.async_copy(xw_h.at[pv.at[pl.ds(kk * 128, 128)]], rbuf,
                         sem).wait()
        pltpu.sync_copy(rbuf,
                        p0_o.at[pl.ds(pl.multiple_of(start + kk * 128, 8),
                                      128)])
        return carry
      lax.fori_loop(0, PC, pbody, 0)

  return k(didx, sidx, perm, xW1, ones_h, zeros_h)


GB = 8


def _sc_dual_pass(A0, A1, init0, init1, ecat, ranges, swap=False):
  (st0, cnt0), (st1, cnt1) = ranges
  GI = 1 if swap else 0
  SI = 1 - GI

  @functools.partial(
      pl.kernel,
      out_type=(
          jax.ShapeDtypeStruct((N, D), jnp.float32),
          jax.ShapeDtypeStruct((N, D), jnp.float32),
      ),
      mesh=_mesh(),
      scratch_types=[
          pltpu.VMEM((2, GB, 2, CH), jnp.int32),
          pltpu.VMEM((2, CH, D), jnp.float32),
          pltpu.VMEM_SHARED((N, D), jnp.float32),
          pltpu.SemaphoreType.DMA,
          pltpu.SemaphoreType.DMA,
          pltpu.SemaphoreType.DMA,
          pltpu.SemaphoreType.DMA,
          pltpu.SemaphoreType.DMA,
      ],
  )
  def k(a0_h, a1_h, i0_h, i1_h, ecat_h, out0_o, out1_o,
        ibg, bufs, acc, sg0, sg1, ss0, ss1, si):
    c = lax.axis_index("c")
    s = lax.axis_index("s")
    start = _tile_row_start(s)
    sg = (sg0, sg1)
    ss = (ss0, ss1)

    @pl.when(c == 0)
    def _():
      pltpu.sync_copy(i0_h.at[pl.ds(start, RPT)], acc.at[pl.ds(start, RPT)])

    @pl.when(c == 1)
    def _():
      pltpu.sync_copy(i1_h.at[pl.ds(start, RPT)], acc.at[pl.ds(start, RPT)])

    plsc.subcore_barrier()

    def run(a_h, st, cnt):
      ngr = cnt // GB

      def wait_scatter(slot, gs_i, u_i):
        pltpu.make_async_copy(bufs.at[slot], acc.at[ibg.at[gs_i, u_i, SI]],
                              ss[slot]).wait()

      pltpu.sync_copy(ecat_h.at[s, pl.ds(st, GB)], ibg.at[0])
      pltpu.async_copy(a_h.at[ibg.at[0, 0, GI]], bufs.at[0], sg[0])

      def group(g, carry):
        gs = g % 2
        base = st + g * GB
        for u in range(GB):
          slot = u % 2
          nslot = (u + 1) % 2
          if u < GB - 1:
            if u >= 1:
              wait_scatter(nslot, gs, u + 1)
            else:
              @pl.when(g > 0)
              def _():
                wait_scatter(nslot, gs, u + 1)
            pltpu.async_copy(a_h.at[ibg.at[gs, u + 1, GI]], bufs.at[nslot],
                             sg[nslot])
          else:
            @pl.when(g + 1 < ngr)
            def _():
              wait_scatter(nslot, 1 - gs, 0)
              pltpu.make_async_copy(ecat_h.at[s, pl.ds(base + GB, GB)],
                                    ibg.at[1 - gs], si).wait()
              pltpu.async_copy(a_h.at[ibg.at[1 - gs, 0, GI]], bufs.at[nslot],
                               sg[nslot])
          pltpu.make_async_copy(a_h.at[ibg.at[gs, u, GI]], bufs.at[slot],
                                sg[slot]).wait()
          pltpu.async_copy(bufs.at[slot], acc.at[ibg.at[gs, u, SI]], ss[slot],
                           add=True)
          if u == 3:
            @pl.when(g + 1 < ngr)
            def _():
              pltpu.async_copy(ecat_h.at[s, pl.ds(base + GB, GB)],
                               ibg.at[1 - gs], si)
        return carry

      lax.fori_loop(0, ngr, group, 0)
      for slot in range(2):
        wait_scatter(slot, 0, slot)

    @pl.when(c == 0)
    def _():
      run(a0_h, st0, cnt0)

    @pl.when(c == 1)
    def _():
      run(a1_h, st1, cnt1)

    plsc.subcore_barrier()

    @pl.when(c == 0)
    def _():
      pltpu.sync_copy(acc.at[pl.ds(start, RPT)],
                      out0_o.at[pl.ds(start, RPT)])

    @pl.when(c == 1)
    def _():
      pltpu.sync_copy(acc.at[pl.ds(start, RPT)],
                      out1_o.at[pl.ds(start, RPT)])

  return k(A0, A1, init0, init1, ecat)


_BLK = 2000


def _row_specs(*widths):
  return [pl.BlockSpec((_BLK, w), lambda i, _w=None: (i, 0)) for w in widths]


def _tc_matmul(x, W):
  def f(x_ref, w_ref, o_ref):
    o_ref[...] = jnp.dot(x_ref[...], w_ref[...],
                         preferred_element_type=jnp.float32)

  return pl.pallas_call(
      f,
      grid=(N // _BLK,),
      in_specs=[
          pl.BlockSpec((_BLK, D), lambda i: (i, 0)),
          pl.BlockSpec((D, D), lambda i: (0, 0)),
      ],
      out_specs=pl.BlockSpec((_BLK, D), lambda i: (i, 0)),
      out_shape=jax.ShapeDtypeStruct((N, D), jnp.float32),
  )(x, W)


def _tc_matmul_scale(x, W, scale):
  def f(x_ref, w_ref, s_ref, o_ref):
    o_ref[...] = jnp.dot(x_ref[...], w_ref[...],
                         preferred_element_type=jnp.float32) * s_ref[...]

  return pl.pallas_call(
      f,
      grid=(N // _BLK,),
      in_specs=[
          pl.BlockSpec((_BLK, D), lambda i: (i, 0)),
          pl.BlockSpec((D, D), lambda i: (0, 0)),
          pl.BlockSpec((_BLK, 1), lambda i: (i, 0)),
      ],
      out_specs=pl.BlockSpec((_BLK, D), lambda i: (i, 0)),
      out_shape=jax.ShapeDtypeStruct((N, D), jnp.float32),
  )(x, W, scale)


def _tc_prescale(hist_d, hist_s, xW1, P0):
  def f(hd_ref, hsr_ref, xw_ref, p0_ref, dis_ref, ci_ref, hs1_ref, hsa_ref):
    deg = hd_ref[...] + 1.0
    dis = lax.rsqrt(deg)
    cnt = hsr_ref[...]
    ci_ref[...] = 1.0 / jnp.where(cnt == 0.0, 1.0, cnt)
    dis_ref[...] = dis
    hs1_ref[...] = xw_ref[...] * dis
    hsa_ref[...] = p0_ref[...] * dis

  return pl.pallas_call(
      f,
      grid=(N // _BLK,),
      in_specs=[
          pl.BlockSpec((_BLK, 1), lambda i: (i, 0)),
          pl.BlockSpec((_BLK, 1), lambda i: (i, 0)),
          pl.BlockSpec((_BLK, D), lambda i: (i, 0)),
          pl.BlockSpec((_BLK, D), lambda i: (i, 0)),
      ],
      out_specs=[
          pl.BlockSpec((_BLK, 1), lambda i: (i, 0)),
          pl.BlockSpec((_BLK, 1), lambda i: (i, 0)),
          pl.BlockSpec((_BLK, D), lambda i: (i, 0)),
          pl.BlockSpec((_BLK, D), lambda i: (i, 0)),
      ],
      out_shape=[
          jax.ShapeDtypeStruct((N, 1), jnp.float32),
          jax.ShapeDtypeStruct((N, 1), jnp.float32),
          jax.ShapeDtypeStruct((N, D), jnp.float32),
          jax.ShapeDtypeStruct((N, D), jnp.float32),
      ],
  )(hist_d, hist_s, xW1, P0)


def _tc_conv_epilogue(acc0, acc1, dis, b):
  def f(a0_ref, a1_ref, dis_ref, b_ref, z_ref, za_ref):
    d = dis_ref[...]
    bb = b_ref[...]
    z_ref[...] = jnp.maximum(a0_ref[...] * d + bb, 0.0)
    za_ref[...] = jnp.maximum(a1_ref[...] * d + bb, 0.0)

  return pl.pallas_call(
      f,
      grid=(N // _BLK,),
      in_specs=[
          pl.BlockSpec((_BLK, D), lambda i: (i, 0)),
          pl.BlockSpec((_BLK, D), lambda i: (i, 0)),
          pl.BlockSpec((_BLK, 1), lambda i: (i, 0)),
          pl.BlockSpec((1, D), lambda i: (0, 0)),
      ],
      out_specs=[
          pl.BlockSpec((_BLK, D), lambda i: (i, 0)),
          pl.BlockSpec((_BLK, D), lambda i: (i, 0)),
      ],
      out_shape=[
          jax.ShapeDtypeStruct((N, D), jnp.float32),
          jax.ShapeDtypeStruct((N, D), jnp.float32),
      ],
  )(acc0, acc1, dis, b)


def _tc_final(c20, c21, r0, r1, z, z_a, cntinv, dis, b2, Wd0, bd):
  def f(c20_ref, c21_ref, r0_ref, r1_ref, z_ref, za_ref, ci_ref, dis_ref,
        b_ref, w_ref, bd_ref, h_ref, ret_ref, reta_ref):
    h_ref[...] = jnp.maximum(
        (c20_ref[...] + c21_ref[...]) * dis_ref[...] + b_ref[...], 0.0)

    def readout(r):
      gr = r * ci_ref[...]
      nrm = jnp.sqrt(jnp.sum(gr * gr, axis=1, keepdims=True))
      gr = gr / jnp.maximum(nrm, 1e-12)
      return 1.0 / (1.0 + jnp.exp(-gr))

    g = readout(r0_ref[...])
    g_a = readout(r1_ref[...])
    wg = lax.dot_general(g, w_ref[...], (((1,), (1,)), ((), ())),
                         preferred_element_type=jnp.float32)
    wga = lax.dot_general(g_a, w_ref[...], (((1,), (1,)), ((), ())),
                          preferred_element_type=jnp.float32)
    b = bd_ref[0, 0]
    s1 = jnp.sum(z_ref[...] * wg, axis=1, keepdims=True)
    s2 = jnp.sum(za_ref[...] * wg, axis=1, keepdims=True)
    ret_ref[...] = jnp.concatenate([s1, s2], axis=1) + b
    s3 = jnp.sum(za_ref[...] * wga, axis=1, keepdims=True)
    s4 = jnp.sum(z_ref[...] * wga, axis=1, keepdims=True)
    reta_ref[...] = jnp.concatenate([s3, s4], axis=1) + b

  return pl.pallas_call(
      f,
      grid=(N // _BLK,),
      in_specs=[
          pl.BlockSpec((_BLK, D), lambda i: (i, 0)),
          pl.BlockSpec((_BLK, D), lambda i: (i, 0)),
          pl.BlockSpec((_BLK, D), lambda i: (i, 0)),
          pl.BlockSpec((_BLK, D), lambda i: (i, 0)),
          pl.BlockSpec((_BLK, D), lambda i: (i, 0)),
          pl.BlockSpec((_BLK, D), lambda i: (i, 0)),
          pl.BlockSpec((_BLK, 1), lambda i: (i, 0)),
          pl.BlockSpec((_BLK, 1), lambda i: (i, 0)),
          pl.BlockSpec((1, D), lambda i: (0, 0)),
          pl.BlockSpec((D, D), lambda i: (0, 0)),
          pl.BlockSpec((1, 1), lambda i: (0, 0)),
      ],
      out_specs=[
          pl.BlockSpec((_BLK, D), lambda i: (i, 0)),
          pl.BlockSpec((_BLK, 2), lambda i: (i, 0)),
          pl.BlockSpec((_BLK, 2), lambda i: (i, 0)),
      ],
      out_shape=[
          jax.ShapeDtypeStruct((N, D), jnp.float32),
          jax.ShapeDtypeStruct((N, 2), jnp.float32),
          jax.ShapeDtypeStruct((N, 2), jnp.float32),
      ],
  )(c20, c21, r0, r1, z, z_a, cntinv, dis, b2, Wd0, bd)


def kernel(x, edge_index, W1, b1, W2, b2, Wd, bd, perm_ids):
  src = edge_index[0].reshape(NS, NCH, CH)
  dst = edge_index[1].reshape(NS, NCH, CH)
  e_conv = jnp.stack([src, dst], axis=2)
  ones_h = jnp.ones((CH,), jnp.float32)
  zeros_h = jnp.zeros((HROW,), jnp.float32)
  zeros_nd = jnp.zeros((N, D), jnp.float32)
  b1r = b1.reshape(1, D)
  b2r = b2.reshape(1, D)
  bdr = bd.reshape(1, 1)

  xW1 = _tc_matmul(x, W1)
  hist, P0 = _sc_hist_perm(dst, src, perm_ids, xW1, ones_h, zeros_h)
  hist_d = hist[0, :N].reshape(N, 1)
  hist_s = hist[1, :N].reshape(N, 1)
  dis, cntinv, hs1, hs1a = _tc_prescale(hist_d, hist_s, xW1, P0)

  c10, c11 = _sc_dual_pass(hs1, hs1a, hs1, hs1a, e_conv,
                           ((0, NCH), (0, NCH)))
  z, z_a = _tc_conv_epilogue(c10, c11, dis, b1r)

  hs2 = _tc_matmul_scale(z, W2, dis)
  r0, r1 = _sc_dual_pass(z, z_a, zeros_nd, zeros_nd, e_conv,
                         ((0, NCH), (0, NCH)), swap=True)
  c20, c21 = _sc_dual_pass(hs2, hs2, hs2, zeros_nd, e_conv,
                           ((0, NCH // 2), (NCH // 2, NCH // 2)))

  h, ret, ret_a = _tc_final(c20, c21, r0, r1, z, z_a, cntinv, dis, b2r,
                            Wd[0], bdr)
  return (z, h, ret, ret_a)


# speedup vs baseline: 21.7195x; 1.0886x over previous
import functools

import jax
import jax.numpy as jnp
from jax import lax
from jax.experimental import pallas as pl
from jax.experimental.pallas import tpu as pltpu
from jax.experimental.pallas import tpu_sc as plsc

N = 10000
E = 320000
D = 128
NC = 2
NS = 16
CH = 100
NCH = E // NS // CH
NCHP = 208
RPT = 640
HROW = 640
HN = NS * HROW
PC = 5


def _tile_row_start(s):
  return pl.multiple_of(jnp.where(s == NS - 1, N - RPT, s * RPT), 8)

_mesh = lambda: plsc.VectorSubcoreMesh(
    core_axis_name="c", subcore_axis_name="s", num_cores=NC, num_subcores=NS)


def _sc_hist_perm(didx, sidx, perm, xW1, ones_h, zeros_h):
  @functools.partial(
      pl.kernel,
      out_type=(
          jax.ShapeDtypeStruct((NC, HN), jnp.float32),
          jax.ShapeDtypeStruct((N, D), jnp.float32),
      ),
      mesh=_mesh(),
      scratch_types=[
          pltpu.VMEM((NCH, CH), jnp.int32),
          pltpu.VMEM((CH,), jnp.float32),
          pltpu.VMEM((RPT,), jnp.int32),
          pltpu.VMEM((128, D), jnp.float32),
          pltpu.VMEM_SHARED((HN,), jnp.float32),
          pltpu.SemaphoreType.DMA,
      ],
  )
  def k(didx_h, sidx_h, perm_h, xw_h, ones_hb, zeros_hb, hist_o, p0_o,
        iv, onesv, pv, rbuf, acc1, sem):
    c = lax.axis_index("c")
    s = lax.axis_index("s")
    pltpu.sync_copy(zeros_hb, acc1.at[pl.ds(s * HROW, HROW)])
    pltpu.sync_copy(ones_hb, onesv)

    @pl.when(c == 0)
    def _():
      pltpu.sync_copy(didx_h.at[s], iv)

    @pl.when(c == 1)
    def _():
      pltpu.sync_copy(sidx_h.at[s], iv)

    plsc.subcore_barrier()

    def body(j, carry):
      pltpu.sync_copy(onesv, acc1.at[iv.at[j]], add=True)
      return carry

    lax.fori_loop(0, NCH, body, 0)
    plsc.subcore_barrier()
    pltpu.sync_copy(acc1.at[pl.ds(s * HROW, HROW)],
                    hist_o.at[c, pl.ds(s * HROW, HROW)])

    @pl.when(c == 1)
    def _():
      start = _tile_row_start(s)
      pltpu.sync_copy(perm_h.at[pl.ds(start, RPT)], pv)
      def pbody(kk, carry):
        pltpu.async_copy(xw_h.at[pv.at[pl.ds(kk * 128, 128)]], rbuf,
                         sem).wait()
        pltpu.sync_copy(rbuf,
                        p0_o.at[pl.ds(pl.multiple_of(start + kk * 128, 8),
                                      128)])
        return carry
      lax.fori_loop(0, PC, pbody, 0)

  return k(didx, sidx, perm, xW1, ones_h, zeros_h)


GB = 12


def _sc_dual_pass(A0, A1, init0, init1, ecat, ranges, swap=False):
  (st0, cnt0), (st1, cnt1) = ranges
  GI = 1 if swap else 0
  SI = 1 - GI

  @functools.partial(
      pl.kernel,
      out_type=(
          jax.ShapeDtypeStruct((N, D), jnp.float32),
          jax.ShapeDtypeStruct((N, D), jnp.float32),
      ),
      mesh=_mesh(),
      scratch_types=[
          pltpu.VMEM((2, GB, 2, CH), jnp.int32),
          pltpu.VMEM((3, CH, D), jnp.float32),
          pltpu.VMEM_SHARED((N, D), jnp.float32),
          pltpu.SemaphoreType.DMA,
          pltpu.SemaphoreType.DMA,
          pltpu.SemaphoreType.DMA,
          pltpu.SemaphoreType.DMA,
          pltpu.SemaphoreType.DMA,
          pltpu.SemaphoreType.DMA,
          pltpu.SemaphoreType.DMA,
      ],
  )
  def k(a0_h, a1_h, i0_h, i1_h, ecat_h, out0_o, out1_o,
        ibg, bufs, acc, sg0, sg1, sg2, ss0, ss1, ss2, si):
    c = lax.axis_index("c")
    s = lax.axis_index("s")
    start = _tile_row_start(s)
    sg = (sg0, sg1, sg2)
    ss = (ss0, ss1, ss2)

    @pl.when(c == 0)
    def _():
      pltpu.sync_copy(i0_h.at[pl.ds(start, RPT)], acc.at[pl.ds(start, RPT)])

    @pl.when(c == 1)
    def _():
      pltpu.sync_copy(i1_h.at[pl.ds(start, RPT)], acc.at[pl.ds(start, RPT)])

    plsc.subcore_barrier()

    def run(a_h, st, cnt):
      ngr = cnt // GB
      rem = cnt % GB

      def wait_scatter(slot):
        pltpu.make_async_copy(bufs.at[slot], acc.at[ibg.at[0, 0, SI]],
                              ss[slot]).wait()

      def gather(slot, gs_i, u_i):
        pltpu.async_copy(a_h.at[ibg.at[gs_i, u_i, GI]], bufs.at[slot],
                         sg[slot])

      def retire(slot, gs_i, u_i):
        pltpu.make_async_copy(a_h.at[ibg.at[gs_i, u_i, GI]], bufs.at[slot],
                              sg[slot]).wait()
        pltpu.async_copy(bufs.at[slot], acc.at[ibg.at[gs_i, u_i, SI]],
                         ss[slot], add=True)

      pltpu.sync_copy(ecat_h.at[s, pl.ds(st, GB)], ibg.at[0])
      gather(0, 0, 0)
      gather(1, 0, 1)

      def maybe_next(g, fn):
        if rem > 0:
          fn()
        else:
          pl.when(g + 1 < ngr)(fn)

      def group(g, carry):
        gs = g % 2
        base = st + g * GB
        for u in range(GB):
          slot = u % 3
          pu = u + 2
          if pu < GB:
            if u >= 1:
              wait_scatter(pu % 3)
            else:
              @pl.when(g > 0)
              def _():
                wait_scatter(pu % 3)
            gather(pu % 3, gs, pu)
          else:
            def _pref(pu=pu, gs=gs, base=base):
              wait_scatter(pu % 3)
              if pu == GB:
                pltpu.make_async_copy(ecat_h.at[s, pl.ds(base + GB, GB)],
                                      ibg.at[1 - gs], si).wait()
              gather(pu % 3, 1 - gs, pu - GB)
            maybe_next(g, _pref)
          retire(slot, gs, u)
          if u == 3:
            def _pidx(gs=gs, base=base):
              pltpu.async_copy(ecat_h.at[s, pl.ds(base + GB, GB)],
                               ibg.at[1 - gs], si)
            maybe_next(g, _pidx)
        return carry

      lax.fori_loop(0, ngr, group, 0)

      if rem > 0:
        egs = ngr % 2
        for u in range(rem):
          pu = u + 2
          if pu < rem:
            wait_scatter(pu % 3)
            gather(pu % 3, egs, pu)
          retire(u % 3, egs, u)
      for slot in range(3):
        wait_scatter(slot)

    @pl.when(c == 0)
    def _():
      run(a0_h, st0, cnt0)

    @pl.when(c == 1)
    def _():
      run(a1_h, st1, cnt1)

    plsc.subcore_barrier()

    @pl.when(c == 0)
    def _():
      pltpu.sync_copy(acc.at[pl.ds(start, RPT)],
                      out0_o.at[pl.ds(start, RPT)])

    @pl.when(c == 1)
    def _():
      pltpu.sync_copy(acc.at[pl.ds(start, RPT)],
                      out1_o.at[pl.ds(start, RPT)])

  return k(A0, A1, init0, init1, ecat)


_BLK = 2000


def _row_specs(*widths):
  return [pl.BlockSpec((_BLK, w), lambda i, _w=None: (i, 0)) for w in widths]


def _tc_matmul(x, W):
  def f(x_ref, w_ref, o_ref):
    o_ref[...] = jnp.dot(x_ref[...], w_ref[...],
                         preferred_element_type=jnp.float32)

  return pl.pallas_call(
      f,
      grid=(N // _BLK,),
      in_specs=[
          pl.BlockSpec((_BLK, D), lambda i: (i, 0)),
          pl.BlockSpec((D, D), lambda i: (0, 0)),
      ],
      out_specs=pl.BlockSpec((_BLK, D), lambda i: (i, 0)),
      out_shape=jax.ShapeDtypeStruct((N, D), jnp.float32),
  )(x, W)


def _tc_matmul_scale(x, W, scale):
  def f(x_ref, w_ref, s_ref, o_ref):
    o_ref[...] = jnp.dot(x_ref[...], w_ref[...],
                         preferred_element_type=jnp.float32) * s_ref[...]

  return pl.pallas_call(
      f,
      grid=(N // _BLK,),
      in_specs=[
          pl.BlockSpec((_BLK, D), lambda i: (i, 0)),
          pl.BlockSpec((D, D), lambda i: (0, 0)),
          pl.BlockSpec((_BLK, 1), lambda i: (i, 0)),
      ],
      out_specs=pl.BlockSpec((_BLK, D), lambda i: (i, 0)),
      out_shape=jax.ShapeDtypeStruct((N, D), jnp.float32),
  )(x, W, scale)


def _tc_prescale(hist_d, hist_s, xW1, P0):
  def f(hd_ref, hsr_ref, xw_ref, p0_ref, dis_ref, ci_ref, hs1_ref, hsa_ref):
    deg = hd_ref[...] + 1.0
    dis = lax.rsqrt(deg)
    cnt = hsr_ref[...]
    ci_ref[...] = 1.0 / jnp.where(cnt == 0.0, 1.0, cnt)
    dis_ref[...] = dis
    hs1_ref[...] = xw_ref[...] * dis
    hsa_ref[...] = p0_ref[...] * dis

  return pl.pallas_call(
      f,
      grid=(N // _BLK,),
      in_specs=[
          pl.BlockSpec((_BLK, 1), lambda i: (i, 0)),
          pl.BlockSpec((_BLK, 1), lambda i: (i, 0)),
          pl.BlockSpec((_BLK, D), lambda i: (i, 0)),
          pl.BlockSpec((_BLK, D), lambda i: (i, 0)),
      ],
      out_specs=[
          pl.BlockSpec((_BLK, 1), lambda i: (i, 0)),
          pl.BlockSpec((_BLK, 1), lambda i: (i, 0)),
          pl.BlockSpec((_BLK, D), lambda i: (i, 0)),
          pl.BlockSpec((_BLK, D), lambda i: (i, 0)),
      ],
      out_shape=[
          jax.ShapeDtypeStruct((N, 1), jnp.float32),
          jax.ShapeDtypeStruct((N, 1), jnp.float32),
          jax.ShapeDtypeStruct((N, D), jnp.float32),
          jax.ShapeDtypeStruct((N, D), jnp.float32),
      ],
  )(hist_d, hist_s, xW1, P0)


def _tc_conv_epilogue(acc0, acc1, dis, b):
  def f(a0_ref, a1_ref, dis_ref, b_ref, z_ref, za_ref):
    d = dis_ref[...]
    bb = b_ref[...]
    z_ref[...] = jnp.maximum(a0_ref[...] * d + bb, 0.0)
    za_ref[...] = jnp.maximum(a1_ref[...] * d + bb, 0.0)

  return pl.pallas_call(
      f,
      grid=(N // _BLK,),
      in_specs=[
          pl.BlockSpec((_BLK, D), lambda i: (i, 0)),
          pl.BlockSpec((_BLK, D), lambda i: (i, 0)),
          pl.BlockSpec((_BLK, 1), lambda i: (i, 0)),
          pl.BlockSpec((1, D), lambda i: (0, 0)),
      ],
      out_specs=[
          pl.BlockSpec((_BLK, D), lambda i: (i, 0)),
          pl.BlockSpec((_BLK, D), lambda i: (i, 0)),
      ],
      out_shape=[
          jax.ShapeDtypeStruct((N, D), jnp.float32),
          jax.ShapeDtypeStruct((N, D), jnp.float32),
      ],
  )(acc0, acc1, dis, b)


def _tc_final(c20, c21, r0, r1, z, z_a, cntinv, dis, b2, Wd0, bd):
  def f(c20_ref, c21_ref, r0_ref, r1_ref, z_ref, za_ref, ci_ref, dis_ref,
        b_ref, w_ref, bd_ref, h_ref, ret_ref, reta_ref):
    h_ref[...] = jnp.maximum(
        (c20_ref[...] + c21_ref[...]) * dis_ref[...] + b_ref[...], 0.0)

    def readout(r):
      gr = r * ci_ref[...]
      nrm = jnp.sqrt(jnp.sum(gr * gr, axis=1, keepdims=True))
      gr = gr / jnp.maximum(nrm, 1e-12)
      return 1.0 / (1.0 + jnp.exp(-gr))

    g = readout(r0_ref[...])
    g_a = readout(r1_ref[...])
    wg = lax.dot_general(g, w_ref[...], (((1,), (1,)), ((), ())),
                         preferred_element_type=jnp.float32)
    wga = lax.dot_general(g_a, w_ref[...], (((1,), (1,)), ((), ())),
                          preferred_element_type=jnp.float32)
    b = bd_ref[0, 0]
    s1 = jnp.sum(z_ref[...] * wg, axis=1, keepdims=True)
    s2 = jnp.sum(za_ref[...] * wg, axis=1, keepdims=True)
    ret_ref[...] = jnp.concatenate([s1, s2], axis=1) + b
    s3 = jnp.sum(za_ref[...] * wga, axis=1, keepdims=True)
    s4 = jnp.sum(z_ref[...] * wga, axis=1, keepdims=True)
    reta_ref[...] = jnp.concatenate([s3, s4], axis=1) + b

  return pl.pallas_call(
      f,
      grid=(N // _BLK,),
      in_specs=[
          pl.BlockSpec((_BLK, D), lambda i: (i, 0)),
          pl.BlockSpec((_BLK, D), lambda i: (i, 0)),
          pl.BlockSpec((_BLK, D), lambda i: (i, 0)),
          pl.BlockSpec((_BLK, D), lambda i: (i, 0)),
          pl.BlockSpec((_BLK, D), lambda i: (i, 0)),
          pl.BlockSpec((_BLK, D), lambda i: (i, 0)),
          pl.BlockSpec((_BLK, 1), lambda i: (i, 0)),
          pl.BlockSpec((_BLK, 1), lambda i: (i, 0)),
          pl.BlockSpec((1, D), lambda i: (0, 0)),
          pl.BlockSpec((D, D), lambda i: (0, 0)),
          pl.BlockSpec((1, 1), lambda i: (0, 0)),
      ],
      out_specs=[
          pl.BlockSpec((_BLK, D), lambda i: (i, 0)),
          pl.BlockSpec((_BLK, 2), lambda i: (i, 0)),
          pl.BlockSpec((_BLK, 2), lambda i: (i, 0)),
      ],
      out_shape=[
          jax.ShapeDtypeStruct((N, D), jnp.float32),
          jax.ShapeDtypeStruct((N, 2), jnp.float32),
          jax.ShapeDtypeStruct((N, 2), jnp.float32),
      ],
  )(c20, c21, r0, r1, z, z_a, cntinv, dis, b2, Wd0, bd)


def kernel(x, edge_index, W1, b1, W2, b2, Wd, bd, perm_ids):
  src = edge_index[0].reshape(NS, NCH, CH)
  dst = edge_index[1].reshape(NS, NCH, CH)
  e_conv = jnp.stack([src, dst], axis=2)
  e_conv = jnp.pad(e_conv, ((0, 0), (0, NCHP - NCH), (0, 0), (0, 0)))
  ones_h = jnp.ones((CH,), jnp.float32)
  zeros_h = jnp.zeros((HROW,), jnp.float32)
  zeros_nd = jnp.zeros((N, D), jnp.float32)
  b1r = b1.reshape(1, D)
  b2r = b2.reshape(1, D)
  bdr = bd.reshape(1, 1)

  xW1 = _tc_matmul(x, W1)
  hist, P0 = _sc_hist_perm(dst, src, perm_ids, xW1, ones_h, zeros_h)
  hist_d = hist[0, :N].reshape(N, 1)
  hist_s = hist[1, :N].reshape(N, 1)
  dis, cntinv, hs1, hs1a = _tc_prescale(hist_d, hist_s, xW1, P0)

  c10, c11 = _sc_dual_pass(hs1, hs1a, hs1, hs1a, e_conv,
                           ((0, NCH), (0, NCH)))
  z, z_a = _tc_conv_epilogue(c10, c11, dis, b1r)

  hs2 = _tc_matmul_scale(z, W2, dis)
  r0, r1 = _sc_dual_pass(z, z_a, zeros_nd, zeros_nd, e_conv,
                         ((0, NCH), (0, NCH)), swap=True)
  c20, c21 = _sc_dual_pass(hs2, hs2, hs2, zeros_nd, e_conv,
                           ((0, NCH // 2), (NCH // 2, NCH // 2)))

  h, ret, ret_a = _tc_final(c20, c21, r0, r1, z, z_a, cntinv, dis, b2r,
                            Wd[0], bdr)
  return (z, h, ret, ret_a)


# speedup vs baseline: 22.1692x; 1.0207x over previous
import functools

import jax
import jax.numpy as jnp
from jax import lax
from jax.experimental import pallas as pl
from jax.experimental.pallas import tpu as pltpu
from jax.experimental.pallas import tpu_sc as plsc

N = 10000
E = 320000
D = 128
NC = 2
NS = 16
CH = 80
NCH = E // NS // CH
NCHP = 264
NSLOT = 4
PD = NSLOT - 1
RPT = 640
HROW = 640
HN = NS * HROW
PC = 5


def _tile_row_start(s):
  return pl.multiple_of(jnp.where(s == NS - 1, N - RPT, s * RPT), 8)

_mesh = lambda: plsc.VectorSubcoreMesh(
    core_axis_name="c", subcore_axis_name="s", num_cores=NC, num_subcores=NS)


def _sc_hist_perm(didx, sidx, perm, xW1, ones_h, zeros_h):
  @functools.partial(
      pl.kernel,
      out_type=(
          jax.ShapeDtypeStruct((NC, HN), jnp.float32),
          jax.ShapeDtypeStruct((N, D), jnp.float32),
      ),
      mesh=_mesh(),
      scratch_types=[
          pltpu.VMEM((NCH, CH), jnp.int32),
          pltpu.VMEM((CH,), jnp.float32),
          pltpu.VMEM((RPT,), jnp.int32),
          pltpu.VMEM((128, D), jnp.float32),
          pltpu.VMEM_SHARED((HN,), jnp.float32),
          pltpu.SemaphoreType.DMA,
      ],
  )
  def k(didx_h, sidx_h, perm_h, xw_h, ones_hb, zeros_hb, hist_o, p0_o,
        iv, onesv, pv, rbuf, acc1, sem):
    c = lax.axis_index("c")
    s = lax.axis_index("s")
    pltpu.sync_copy(zeros_hb, acc1.at[pl.ds(s * HROW, HROW)])
    pltpu.sync_copy(ones_hb, onesv)

    @pl.when(c == 0)
    def _():
      pltpu.sync_copy(didx_h.at[s], iv)

    @pl.when(c == 1)
    def _():
      pltpu.sync_copy(sidx_h.at[s], iv)

    plsc.subcore_barrier()

    def body(j, carry):
      pltpu.sync_copy(onesv, acc1.at[iv.at[j]], add=True)
      return carry

    lax.fori_loop(0, NCH, body, 0)
    plsc.subcore_barrier()
    pltpu.sync_copy(acc1.at[pl.ds(s * HROW, HROW)],
                    hist_o.at[c, pl.ds(s * HROW, HROW)])

    @pl.when(c == 1)
    def _():
      start = _tile_row_start(s)
      pltpu.sync_copy(perm_h.at[pl.ds(start, RPT)], pv)
      def pbody(kk, carry):
        pltpu.async_copy(xw_h.at[pv.at[pl.ds(kk * 128, 128)]], rbuf,
                         sem).wait()
        pltpu.sync_copy(rbuf,
                        p0_o.at[pl.ds(pl.multiple_of(start + kk * 128, 8),
                                      128)])
        return carry
      lax.fori_loop(0, PC, pbody, 0)

  return k(didx, sidx, perm, xW1, ones_h, zeros_h)


GB = 12


def _sc_dual_pass(A0, A1, init0, init1, ecat, ranges, swap=False):
  (st0, cnt0), (st1, cnt1) = ranges
  GI = 1 if swap else 0
  SI = 1 - GI

  @functools.partial(
      pl.kernel,
      out_type=(
          jax.ShapeDtypeStruct((N, D), jnp.float32),
          jax.ShapeDtypeStruct((N, D), jnp.float32),
      ),
      mesh=_mesh(),
      scratch_types=[
          pltpu.VMEM((2, GB, 2, CH), jnp.int32),
          pltpu.VMEM((NSLOT, CH, D), jnp.float32),
          pltpu.VMEM_SHARED((N, D), jnp.float32),
          pltpu.SemaphoreType.DMA,
          pltpu.SemaphoreType.DMA,
          pltpu.SemaphoreType.DMA,
          pltpu.SemaphoreType.DMA,
          pltpu.SemaphoreType.DMA,
          pltpu.SemaphoreType.DMA,
          pltpu.SemaphoreType.DMA,
          pltpu.SemaphoreType.DMA,
          pltpu.SemaphoreType.DMA,
      ],
  )
  def k(a0_h, a1_h, i0_h, i1_h, ecat_h, out0_o, out1_o,
        ibg, bufs, acc, sg0, sg1, sg2, sg3, ss0, ss1, ss2, ss3, si):
    c = lax.axis_index("c")
    s = lax.axis_index("s")
    start = _tile_row_start(s)
    sg = (sg0, sg1, sg2, sg3)
    ss = (ss0, ss1, ss2, ss3)

    @pl.when(c == 0)
    def _():
      pltpu.sync_copy(i0_h.at[pl.ds(start, RPT)], acc.at[pl.ds(start, RPT)])

    @pl.when(c == 1)
    def _():
      pltpu.sync_copy(i1_h.at[pl.ds(start, RPT)], acc.at[pl.ds(start, RPT)])

    plsc.subcore_barrier()

    def run(a_h, st, cnt):
      ngr = cnt // GB
      rem = cnt % GB
      assert rem == 0 or rem >= PD

      def wait_scatter(slot):
        pltpu.make_async_copy(bufs.at[slot], acc.at[ibg.at[0, 0, SI]],
                              ss[slot]).wait()

      def gather(slot, gs_i, u_i):
        pltpu.async_copy(a_h.at[ibg.at[gs_i, u_i, GI]], bufs.at[slot],
                         sg[slot])

      def retire(slot, gs_i, u_i):
        pltpu.make_async_copy(a_h.at[ibg.at[gs_i, u_i, GI]], bufs.at[slot],
                              sg[slot]).wait()
        pltpu.async_copy(bufs.at[slot], acc.at[ibg.at[gs_i, u_i, SI]],
                         ss[slot], add=True)

      pltpu.sync_copy(ecat_h.at[s, pl.ds(st, GB)], ibg.at[0])
      for p in range(PD):
        gather(p, 0, p)

      def maybe_next(g, fn):
        if rem > 0:
          fn()
        else:
          pl.when(g + 1 < ngr)(fn)

      def group(g, carry):
        gs = g % 2
        base = st + g * GB
        for u in range(GB):
          slot = u % NSLOT
          pu = u + PD
          if pu < GB:
            if u >= 1:
              wait_scatter(pu % NSLOT)
            else:
              @pl.when(g > 0)
              def _():
                wait_scatter(pu % NSLOT)
            gather(pu % NSLOT, gs, pu)
          else:
            def _pref(pu=pu, gs=gs, base=base):
              wait_scatter(pu % NSLOT)
              if pu == GB:
                pltpu.make_async_copy(ecat_h.at[s, pl.ds(base + GB, GB)],
                                      ibg.at[1 - gs], si).wait()
              gather(pu % NSLOT, 1 - gs, pu - GB)
            maybe_next(g, _pref)
          retire(slot, gs, u)
          if u == 3:
            def _pidx(gs=gs, base=base):
              pltpu.async_copy(ecat_h.at[s, pl.ds(base + GB, GB)],
                               ibg.at[1 - gs], si)
            maybe_next(g, _pidx)
        return carry

      lax.fori_loop(0, ngr, group, 0)

      if rem > 0:
        egs = ngr % 2
        for u in range(rem):
          pu = u + PD
          if pu < rem:
            wait_scatter(pu % NSLOT)
            gather(pu % NSLOT, egs, pu)
          retire(u % NSLOT, egs, u)
      for slot in range(NSLOT):
        wait_scatter(slot)

    @pl.when(c == 0)
    def _():
      run(a0_h, st0, cnt0)

    @pl.when(c == 1)
    def _():
      run(a1_h, st1, cnt1)

    plsc.subcore_barrier()

    @pl.when(c == 0)
    def _():
      pltpu.sync_copy(acc.at[pl.ds(start, RPT)],
                      out0_o.at[pl.ds(start, RPT)])

    @pl.when(c == 1)
    def _():
      pltpu.sync_copy(acc.at[pl.ds(start, RPT)],
                      out1_o.at[pl.ds(start, RPT)])

  return k(A0, A1, init0, init1, ecat)


_BLK = 2000


def _row_specs(*widths):
  return [pl.BlockSpec((_BLK, w), lambda i, _w=None: (i, 0)) for w in widths]


def _tc_matmul(x, W):
  def f(x_ref, w_ref, o_ref):
    o_ref[...] = jnp.dot(x_ref[...], w_ref[...],
                         preferred_element_type=jnp.float32)

  return pl.pallas_call(
      f,
      grid=(N // _BLK,),
      in_specs=[
          pl.BlockSpec((_BLK, D), lambda i: (i, 0)),
          pl.BlockSpec((D, D), lambda i: (0, 0)),
      ],
      out_specs=pl.BlockSpec((_BLK, D), lambda i: (i, 0)),
      out_shape=jax.ShapeDtypeStruct((N, D), jnp.float32),
  )(x, W)


def _tc_matmul_scale(x, W, scale):
  def f(x_ref, w_ref, s_ref, o_ref):
    o_ref[...] = jnp.dot(x_ref[...], w_ref[...],
                         preferred_element_type=jnp.float32) * s_ref[...]

  return pl.pallas_call(
      f,
      grid=(N // _BLK,),
      in_specs=[
          pl.BlockSpec((_BLK, D), lambda i: (i, 0)),
          pl.BlockSpec((D, D), lambda i: (0, 0)),
          pl.BlockSpec((_BLK, 1), lambda i: (i, 0)),
      ],
      out_specs=pl.BlockSpec((_BLK, D), lambda i: (i, 0)),
      out_shape=jax.ShapeDtypeStruct((N, D), jnp.float32),
  )(x, W, scale)


def _tc_prescale(hist_d, hist_s, xW1, P0):
  def f(hd_ref, hsr_ref, xw_ref, p0_ref, dis_ref, ci_ref, hs1_ref, hsa_ref):
    deg = hd_ref[...] + 1.0
    dis = lax.rsqrt(deg)
    cnt = hsr_ref[...]
    ci_ref[...] = 1.0 / jnp.where(cnt == 0.0, 1.0, cnt)
    dis_ref[...] = dis
    hs1_ref[...] = xw_ref[...] * dis
    hsa_ref[...] = p0_ref[...] * dis

  return pl.pallas_call(
      f,
      grid=(N // _BLK,),
      in_specs=[
          pl.BlockSpec((_BLK, 1), lambda i: (i, 0)),
          pl.BlockSpec((_BLK, 1), lambda i: (i, 0)),
          pl.BlockSpec((_BLK, D), lambda i: (i, 0)),
          pl.BlockSpec((_BLK, D), lambda i: (i, 0)),
      ],
      out_specs=[
          pl.BlockSpec((_BLK, 1), lambda i: (i, 0)),
          pl.BlockSpec((_BLK, 1), lambda i: (i, 0)),
          pl.BlockSpec((_BLK, D), lambda i: (i, 0)),
          pl.BlockSpec((_BLK, D), lambda i: (i, 0)),
      ],
      out_shape=[
          jax.ShapeDtypeStruct((N, 1), jnp.float32),
          jax.ShapeDtypeStruct((N, 1), jnp.float32),
          jax.ShapeDtypeStruct((N, D), jnp.float32),
          jax.ShapeDtypeStruct((N, D), jnp.float32),
      ],
  )(hist_d, hist_s, xW1, P0)


def _tc_conv_epilogue(acc0, acc1, dis, b):
  def f(a0_ref, a1_ref, dis_ref, b_ref, z_ref, za_ref):
    d = dis_ref[...]
    bb = b_ref[...]
    z_ref[...] = jnp.maximum(a0_ref[...] * d + bb, 0.0)
    za_ref[...] = jnp.maximum(a1_ref[...] * d + bb, 0.0)

  return pl.pallas_call(
      f,
      grid=(N // _BLK,),
      in_specs=[
          pl.BlockSpec((_BLK, D), lambda i: (i, 0)),
          pl.BlockSpec((_BLK, D), lambda i: (i, 0)),
          pl.BlockSpec((_BLK, 1), lambda i: (i, 0)),
          pl.BlockSpec((1, D), lambda i: (0, 0)),
      ],
      out_specs=[
          pl.BlockSpec((_BLK, D), lambda i: (i, 0)),
          pl.BlockSpec((_BLK, D), lambda i: (i, 0)),
      ],
      out_shape=[
          jax.ShapeDtypeStruct((N, D), jnp.float32),
          jax.ShapeDtypeStruct((N, D), jnp.float32),
      ],
  )(acc0, acc1, dis, b)


def _tc_final(c20, c21, r0, r1, z, z_a, cntinv, dis, b2, Wd0, bd):
  def f(c20_ref, c21_ref, r0_ref, r1_ref, z_ref, za_ref, ci_ref, dis_ref,
        b_ref, w_ref, bd_ref, h_ref, ret_ref, reta_ref):
    h_ref[...] = jnp.maximum(
        (c20_ref[...] + c21_ref[...]) * dis_ref[...] + b_ref[...], 0.0)

    def readout(r):
      gr = r * ci_ref[...]
      nrm = jnp.sqrt(jnp.sum(gr * gr, axis=1, keepdims=True))
      gr = gr / jnp.maximum(nrm, 1e-12)
      return 1.0 / (1.0 + jnp.exp(-gr))

    g = readout(r0_ref[...])
    g_a = readout(r1_ref[...])
    wg = lax.dot_general(g, w_ref[...], (((1,), (1,)), ((), ())),
                         preferred_element_type=jnp.float32)
    wga = lax.dot_general(g_a, w_ref[...], (((1,), (1,)), ((), ())),
                          preferred_element_type=jnp.float32)
    b = bd_ref[0, 0]
    s1 = jnp.sum(z_ref[...] * wg, axis=1, keepdims=True)
    s2 = jnp.sum(za_ref[...] * wg, axis=1, keepdims=True)
    ret_ref[...] = jnp.concatenate([s1, s2], axis=1) + b
    s3 = jnp.sum(za_ref[...] * wga, axis=1, keepdims=True)
    s4 = jnp.sum(z_ref[...] * wga, axis=1, keepdims=True)
    reta_ref[...] = jnp.concatenate([s3, s4], axis=1) + b

  return pl.pallas_call(
      f,
      grid=(N // _BLK,),
      in_specs=[
          pl.BlockSpec((_BLK, D), lambda i: (i, 0)),
          pl.BlockSpec((_BLK, D), lambda i: (i, 0)),
          pl.BlockSpec((_BLK, D), lambda i: (i, 0)),
          pl.BlockSpec((_BLK, D), lambda i: (i, 0)),
          pl.BlockSpec((_BLK, D), lambda i: (i, 0)),
          pl.BlockSpec((_BLK, D), lambda i: (i, 0)),
          pl.BlockSpec((_BLK, 1), lambda i: (i, 0)),
          pl.BlockSpec((_BLK, 1), lambda i: (i, 0)),
          pl.BlockSpec((1, D), lambda i: (0, 0)),
          pl.BlockSpec((D, D), lambda i: (0, 0)),
          pl.BlockSpec((1, 1), lambda i: (0, 0)),
      ],
      out_specs=[
          pl.BlockSpec((_BLK, D), lambda i: (i, 0)),
          pl.BlockSpec((_BLK, 2), lambda i: (i, 0)),
          pl.BlockSpec((_BLK, 2), lambda i: (i, 0)),
      ],
      out_shape=[
          jax.ShapeDtypeStruct((N, D), jnp.float32),
          jax.ShapeDtypeStruct((N, 2), jnp.float32),
          jax.ShapeDtypeStruct((N, 2), jnp.float32),
      ],
  )(c20, c21, r0, r1, z, z_a, cntinv, dis, b2, Wd0, bd)


def kernel(x, edge_index, W1, b1, W2, b2, Wd, bd, perm_ids):
  src = edge_index[0].reshape(NS, NCH, CH)
  dst = edge_index[1].reshape(NS, NCH, CH)
  e_conv = jnp.stack([src, dst], axis=2)
  e_conv = jnp.pad(e_conv, ((0, 0), (0, NCHP - NCH), (0, 0), (0, 0)))
  ones_h = jnp.ones((CH,), jnp.float32)
  zeros_h = jnp.zeros((HROW,), jnp.float32)
  zeros_nd = jnp.zeros((N, D), jnp.float32)
  b1r = b1.reshape(1, D)
  b2r = b2.reshape(1, D)
  bdr = bd.reshape(1, 1)

  xW1 = _tc_matmul(x, W1)
  hist, P0 = _sc_hist_perm(dst, src, perm_ids, xW1, ones_h, zeros_h)
  hist_d = hist[0, :N].reshape(N, 1)
  hist_s = hist[1, :N].reshape(N, 1)
  dis, cntinv, hs1, hs1a = _tc_prescale(hist_d, hist_s, xW1, P0)

  c10, c11 = _sc_dual_pass(hs1, hs1a, hs1, hs1a, e_conv,
                           ((0, NCH), (0, NCH)))
  z, z_a = _tc_conv_epilogue(c10, c11, dis, b1r)

  hs2 = _tc_matmul_scale(z, W2, dis)
  r0, r1 = _sc_dual_pass(z, z_a, zeros_nd, zeros_nd, e_conv,
                         ((0, NCH), (0, NCH)), swap=True)
  c20, c21 = _sc_dual_pass(hs2, hs2, hs2, zeros_nd, e_conv,
                           ((0, NCH // 2), (NCH // 2, NCH // 2)))

  h, ret, ret_a = _tc_final(c20, c21, r0, r1, z, z_a, cntinv, dis, b2r,
                            Wd[0], bdr)
  return (z, h, ret, ret_a)


# speedup vs baseline: 22.4622x; 1.0132x over previous
import functools

import jax
import jax.numpy as jnp
from jax import lax
from jax.experimental import pallas as pl
from jax.experimental.pallas import tpu as pltpu
from jax.experimental.pallas import tpu_sc as plsc

N = 10000
E = 320000
D = 128
NC = 2
NS = 16
CH = 80
NCH = E // NS // CH
NCHP = 264
NSLOT = 4
PD = NSLOT - 1
RPT = 640
HROW = 640
HN = NS * HROW
PC = 5


def _tile_row_start(s):
  return pl.multiple_of(jnp.where(s == NS - 1, N - RPT, s * RPT), 8)

_mesh = lambda: plsc.VectorSubcoreMesh(
    core_axis_name="c", subcore_axis_name="s", num_cores=NC, num_subcores=NS)


def _sc_hist_perm(didx, sidx, perm, xW1, ones_h, zeros_h):
  @functools.partial(
      pl.kernel,
      out_type=(
          jax.ShapeDtypeStruct((NC, HN), jnp.float32),
          jax.ShapeDtypeStruct((N, D), jnp.float32),
      ),
      mesh=_mesh(),
      scratch_types=[
          pltpu.VMEM((NCH, CH), jnp.int32),
          pltpu.VMEM((CH,), jnp.float32),
          pltpu.VMEM((RPT,), jnp.int32),
          pltpu.VMEM((128, D), jnp.float32),
          pltpu.VMEM_SHARED((HN,), jnp.float32),
          pltpu.SemaphoreType.DMA,
      ],
  )
  def k(didx_h, sidx_h, perm_h, xw_h, ones_hb, zeros_hb, hist_o, p0_o,
        iv, onesv, pv, rbuf, acc1, sem):
    c = lax.axis_index("c")
    s = lax.axis_index("s")
    pltpu.sync_copy(zeros_hb, acc1.at[pl.ds(s * HROW, HROW)])
    pltpu.sync_copy(ones_hb, onesv)

    @pl.when(c == 0)
    def _():
      pltpu.sync_copy(didx_h.at[s], iv)

    @pl.when(c == 1)
    def _():
      pltpu.sync_copy(sidx_h.at[s], iv)

    plsc.subcore_barrier()

    def body(j, carry):
      pltpu.sync_copy(onesv, acc1.at[iv.at[j]], add=True)
      return carry

    lax.fori_loop(0, NCH, body, 0)
    plsc.subcore_barrier()
    pltpu.sync_copy(acc1.at[pl.ds(s * HROW, HROW)],
                    hist_o.at[c, pl.ds(s * HROW, HROW)])

    @pl.when(c == 1)
    def _():
      start = _tile_row_start(s)
      pltpu.sync_copy(perm_h.at[pl.ds(start, RPT)], pv)
      def pbody(kk, carry):
        pltpu.async_copy(xw_h.at[pv.at[pl.ds(kk * 128, 128)]], rbuf,
                         sem).wait()
        pltpu.sync_copy(rbuf,
                        p0_o.at[pl.ds(pl.multiple_of(start + kk * 128, 8),
                                      128)])
        return carry
      lax.fori_loop(0, PC, pbody, 0)

  return k(didx, sidx, perm, xW1, ones_h, zeros_h)


GB = 12


def _sc_dual_pass(A0, A1, init0, init1, ecat, ranges, swap=False):
  (st0, cnt0), (st1, cnt1) = ranges
  GI = 1 if swap else 0
  SI = 1 - GI

  @functools.partial(
      pl.kernel,
      out_type=(
          jax.ShapeDtypeStruct((N, D), jnp.float32),
          jax.ShapeDtypeStruct((N, D), jnp.float32),
      ),
      mesh=_mesh(),
      scratch_types=[
          pltpu.VMEM((2, GB, 2, CH), jnp.int32),
          pltpu.VMEM((NSLOT, CH, D), jnp.float32),
          pltpu.VMEM_SHARED((N, D), jnp.float32),
          pltpu.SemaphoreType.DMA,
          pltpu.SemaphoreType.DMA,
          pltpu.SemaphoreType.DMA,
          pltpu.SemaphoreType.DMA,
          pltpu.SemaphoreType.DMA,
          pltpu.SemaphoreType.DMA,
          pltpu.SemaphoreType.DMA,
          pltpu.SemaphoreType.DMA,
          pltpu.SemaphoreType.DMA,
      ],
  )
  def k(a0_h, a1_h, i0_h, i1_h, ecat_h, out0_o, out1_o,
        ibg, bufs, acc, sg0, sg1, sg2, sg3, ss0, ss1, ss2, ss3, si):
    c = lax.axis_index("c")
    s = lax.axis_index("s")
    start = _tile_row_start(s)
    sg = (sg0, sg1, sg2, sg3)
    ss = (ss0, ss1, ss2, ss3)

    @pl.when(c == 0)
    def _():
      pltpu.sync_copy(i0_h.at[pl.ds(start, RPT)], acc.at[pl.ds(start, RPT)])

    @pl.when(c == 1)
    def _():
      pltpu.sync_copy(i1_h.at[pl.ds(start, RPT)], acc.at[pl.ds(start, RPT)])

    plsc.subcore_barrier()

    def run(a_h, st, cnt):
      ngr = cnt // GB
      rem = cnt % GB
      assert rem == 0 or rem >= PD

      def wait_scatter(slot):
        pltpu.make_async_copy(bufs.at[slot], acc.at[ibg.at[0, 0, SI]],
                              ss[slot]).wait()

      def gather(slot, gs_i, u_i):
        pltpu.async_copy(a_h.at[ibg.at[gs_i, u_i, GI]], bufs.at[slot],
                         sg[slot])

      def retire(slot, gs_i, u_i):
        pltpu.make_async_copy(a_h.at[ibg.at[gs_i, u_i, GI]], bufs.at[slot],
                              sg[slot]).wait()
        pltpu.async_copy(bufs.at[slot], acc.at[ibg.at[gs_i, u_i, SI]],
                         ss[slot], add=True)

      pltpu.sync_copy(ecat_h.at[s, pl.ds(st, GB)], ibg.at[0])
      for p in range(PD):
        gather(p, 0, p)

      def maybe_next(g, fn):
        if rem > 0:
          fn()
        else:
          pl.when(g + 1 < ngr)(fn)

      def group(g, carry):
        gs = g % 2
        base = st + g * GB
        for u in range(GB):
          slot = u % NSLOT
          pu = u + PD
          if pu < GB:
            if u >= 1:
              wait_scatter(pu % NSLOT)
            else:
              @pl.when(g > 0)
              def _():
                wait_scatter(pu % NSLOT)
            gather(pu % NSLOT, gs, pu)
          else:
            def _pref(pu=pu, gs=gs, base=base):
              wait_scatter(pu % NSLOT)
              if pu == GB:
                pltpu.make_async_copy(ecat_h.at[s, pl.ds(base + GB, GB)],
                                      ibg.at[1 - gs], si).wait()
              gather(pu % NSLOT, 1 - gs, pu - GB)
            maybe_next(g, _pref)
          retire(slot, gs, u)
          if u == 3:
            def _pidx(gs=gs, base=base):
              pltpu.async_copy(ecat_h.at[s, pl.ds(base + GB, GB)],
                               ibg.at[1 - gs], si)
            maybe_next(g, _pidx)
        return carry

      lax.fori_loop(0, ngr, group, 0)

      if rem > 0:
        egs = ngr % 2
        for u in range(rem):
          pu = u + PD
          if pu < rem:
            wait_scatter(pu % NSLOT)
            gather(pu % NSLOT, egs, pu)
          retire(u % NSLOT, egs, u)
      for slot in range(NSLOT):
        wait_scatter(slot)

    @pl.when(c == 0)
    def _():
      run(a0_h, st0, cnt0)

    @pl.when(c == 1)
    def _():
      run(a1_h, st1, cnt1)

    plsc.subcore_barrier()

    @pl.when(c == 0)
    def _():
      pltpu.sync_copy(acc.at[pl.ds(start, RPT)],
                      out0_o.at[pl.ds(start, RPT)])

    @pl.when(c == 1)
    def _():
      pltpu.sync_copy(acc.at[pl.ds(start, RPT)],
                      out1_o.at[pl.ds(start, RPT)])

  return k(A0, A1, init0, init1, ecat)


_BLK = 2000


def _row_specs(*widths):
  return [pl.BlockSpec((_BLK, w), lambda i, _w=None: (i, 0)) for w in widths]


def _tc_matmul(x, W):
  def f(x_ref, w_ref, o_ref):
    o_ref[...] = jnp.dot(x_ref[...], w_ref[...],
                         preferred_element_type=jnp.float32)

  return pl.pallas_call(
      f,
      grid=(N // _BLK,),
      in_specs=[
          pl.BlockSpec((_BLK, D), lambda i: (i, 0)),
          pl.BlockSpec((D, D), lambda i: (0, 0)),
      ],
      out_specs=pl.BlockSpec((_BLK, D), lambda i: (i, 0)),
      out_shape=jax.ShapeDtypeStruct((N, D), jnp.float32),
  )(x, W)


def _tc_matmul_scale(x, W, scale):
  def f(x_ref, w_ref, s_ref, o_ref):
    o_ref[...] = jnp.dot(x_ref[...], w_ref[...],
                         preferred_element_type=jnp.float32) * s_ref[...]

  return pl.pallas_call(
      f,
      grid=(N // _BLK,),
      in_specs=[
          pl.BlockSpec((_BLK, D), lambda i: (i, 0)),
          pl.BlockSpec((D, D), lambda i: (0, 0)),
          pl.BlockSpec((_BLK, 1), lambda i: (i, 0)),
      ],
      out_specs=pl.BlockSpec((_BLK, D), lambda i: (i, 0)),
      out_shape=jax.ShapeDtypeStruct((N, D), jnp.float32),
  )(x, W, scale)


def _tc_prescale(hist_d, hist_s, xW1, P0):
  def f(hd_ref, hsr_ref, xw_ref, p0_ref, dis_ref, ci_ref, hs1_ref, hsa_ref):
    deg = hd_ref[...] + 1.0
    dis = lax.rsqrt(deg)
    cnt = hsr_ref[...]
    ci_ref[...] = 1.0 / jnp.where(cnt == 0.0, 1.0, cnt)
    dis_ref[...] = dis
    hs1_ref[...] = xw_ref[...] * dis
    hsa_ref[...] = p0_ref[...] * dis

  return pl.pallas_call(
      f,
      grid=(N // _BLK,),
      in_specs=[
          pl.BlockSpec((_BLK, 1), lambda i: (i, 0)),
          pl.BlockSpec((_BLK, 1), lambda i: (i, 0)),
          pl.BlockSpec((_BLK, D), lambda i: (i, 0)),
          pl.BlockSpec((_BLK, D), lambda i: (i, 0)),
      ],
      out_specs=[
          pl.BlockSpec((_BLK, 1), lambda i: (i, 0)),
          pl.BlockSpec((_BLK, 1), lambda i: (i, 0)),
          pl.BlockSpec((_BLK, D), lambda i: (i, 0)),
          pl.BlockSpec((_BLK, D), lambda i: (i, 0)),
      ],
      out_shape=[
          jax.ShapeDtypeStruct((N, 1), jnp.float32),
          jax.ShapeDtypeStruct((N, 1), jnp.float32),
          jax.ShapeDtypeStruct((N, D), jnp.float32),
          jax.ShapeDtypeStruct((N, D), jnp.float32),
      ],
  )(hist_d, hist_s, xW1, P0)


def _tc_conv_epilogue(acc0, acc1, dis, b, W2):
  def f(a0_ref, a1_ref, dis_ref, b_ref, w_ref, z_ref, za_ref, hs2_ref):
    d = dis_ref[...]
    bb = b_ref[...]
    z = jnp.maximum(a0_ref[...] * d + bb, 0.0)
    z_ref[...] = z
    za_ref[...] = jnp.maximum(a1_ref[...] * d + bb, 0.0)
    hs2_ref[...] = jnp.dot(z, w_ref[...],
                           preferred_element_type=jnp.float32) * d

  return pl.pallas_call(
      f,
      grid=(N // _BLK,),
      in_specs=[
          pl.BlockSpec((_BLK, D), lambda i: (i, 0)),
          pl.BlockSpec((_BLK, D), lambda i: (i, 0)),
          pl.BlockSpec((_BLK, 1), lambda i: (i, 0)),
          pl.BlockSpec((1, D), lambda i: (0, 0)),
          pl.BlockSpec((D, D), lambda i: (0, 0)),
      ],
      out_specs=[
          pl.BlockSpec((_BLK, D), lambda i: (i, 0)),
          pl.BlockSpec((_BLK, D), lambda i: (i, 0)),
          pl.BlockSpec((_BLK, D), lambda i: (i, 0)),
      ],
      out_shape=[
          jax.ShapeDtypeStruct((N, D), jnp.float32),
          jax.ShapeDtypeStruct((N, D), jnp.float32),
          jax.ShapeDtypeStruct((N, D), jnp.float32),
      ],
  )(acc0, acc1, dis, b, W2)


def _tc_final(c20, c21, r0, r1, z, z_a, cntinv, dis, b2, Wd0, bd):
  def f(c20_ref, c21_ref, r0_ref, r1_ref, z_ref, za_ref, ci_ref, dis_ref,
        b_ref, w_ref, bd_ref, h_ref, ret_ref, reta_ref):
    h_ref[...] = jnp.maximum(
        (c20_ref[...] + c21_ref[...]) * dis_ref[...] + b_ref[...], 0.0)

    def readout(r):
      gr = r * ci_ref[...]
      nrm = jnp.sqrt(jnp.sum(gr * gr, axis=1, keepdims=True))
      gr = gr / jnp.maximum(nrm, 1e-12)
      return 1.0 / (1.0 + jnp.exp(-gr))

    g = readout(r0_ref[...])
    g_a = readout(r1_ref[...])
    wg = lax.dot_general(g, w_ref[...], (((1,), (1,)), ((), ())),
                         preferred_element_type=jnp.float32)
    wga = lax.dot_general(g_a, w_ref[...], (((1,), (1,)), ((), ())),
                          preferred_element_type=jnp.float32)
    b = bd_ref[0, 0]
    s1 = jnp.sum(z_ref[...] * wg, axis=1, keepdims=True)
    s2 = jnp.sum(za_ref[...] * wg, axis=1, keepdims=True)
    ret_ref[...] = jnp.concatenate([s1, s2], axis=1) + b
    s3 = jnp.sum(za_ref[...] * wga, axis=1, keepdims=True)
    s4 = jnp.sum(z_ref[...] * wga, axis=1, keepdims=True)
    reta_ref[...] = jnp.concatenate([s3, s4], axis=1) + b

  return pl.pallas_call(
      f,
      grid=(N // _BLK,),
      in_specs=[
          pl.BlockSpec((_BLK, D), lambda i: (i, 0)),
          pl.BlockSpec((_BLK, D), lambda i: (i, 0)),
          pl.BlockSpec((_BLK, D), lambda i: (i, 0)),
          pl.BlockSpec((_BLK, D), lambda i: (i, 0)),
          pl.BlockSpec((_BLK, D), lambda i: (i, 0)),
          pl.BlockSpec((_BLK, D), lambda i: (i, 0)),
          pl.BlockSpec((_BLK, 1), lambda i: (i, 0)),
          pl.BlockSpec((_BLK, 1), lambda i: (i, 0)),
          pl.BlockSpec((1, D), lambda i: (0, 0)),
          pl.BlockSpec((D, D), lambda i: (0, 0)),
          pl.BlockSpec((1, 1), lambda i: (0, 0)),
      ],
      out_specs=[
          pl.BlockSpec((_BLK, D), lambda i: (i, 0)),
          pl.BlockSpec((_BLK, 2), lambda i: (i, 0)),
          pl.BlockSpec((_BLK, 2), lambda i: (i, 0)),
      ],
      out_shape=[
          jax.ShapeDtypeStruct((N, D), jnp.float32),
          jax.ShapeDtypeStruct((N, 2), jnp.float32),
          jax.ShapeDtypeStruct((N, 2), jnp.float32),
      ],
  )(c20, c21, r0, r1, z, z_a, cntinv, dis, b2, Wd0, bd)


def kernel(x, edge_index, W1, b1, W2, b2, Wd, bd, perm_ids):
  src = edge_index[0].reshape(NS, NCH, CH)
  dst = edge_index[1].reshape(NS, NCH, CH)
  e_conv = jnp.stack([src, dst], axis=2)
  e_conv = jnp.pad(e_conv, ((0, 0), (0, NCHP - NCH), (0, 0), (0, 0)))
  ones_h = jnp.ones((CH,), jnp.float32)
  zeros_h = jnp.zeros((HROW,), jnp.float32)
  zeros_nd = jnp.zeros((N, D), jnp.float32)
  b1r = b1.reshape(1, D)
  b2r = b2.reshape(1, D)
  bdr = bd.reshape(1, 1)

  xW1 = _tc_matmul(x, W1)
  hist, P0 = _sc_hist_perm(dst, src, perm_ids, xW1, ones_h, zeros_h)
  hist_d = hist[0, :N].reshape(N, 1)
  hist_s = hist[1, :N].reshape(N, 1)
  dis, cntinv, hs1, hs1a = _tc_prescale(hist_d, hist_s, xW1, P0)

  c10, c11 = _sc_dual_pass(hs1, hs1a, hs1, hs1a, e_conv,
                           ((0, NCH), (0, NCH)))
  z, z_a, hs2 = _tc_conv_epilogue(c10, c11, dis, b1r, W2)

  r0, r1 = _sc_dual_pass(z, z_a, zeros_nd, zeros_nd, e_conv,
                         ((0, NCH), (0, NCH)), swap=True)
  c20, c21 = _sc_dual_pass(hs2, hs2, hs2, zeros_nd, e_conv,
                           ((0, NCH // 2), (NCH // 2, NCH // 2)))

  h, ret, ret_a = _tc_final(c20, c21, r0, r1, z, z_a, cntinv, dis, b2r,
                            Wd[0], bdr)
  return (z, h, ret, ret_a)
